# Initial kernel scaffold; baseline (speedup 1.0000x reference)
import jax
import jax.numpy as jnp
from jax.experimental import pallas as pl


def kernel(x, edge_index, edge_type, neg_dst, W_emb, b_emb, bases0, w_coe0, self_loop0, gamma0, beta0, mm0, mv0, bases1, w_coe1, self_loop1, gamma1, beta1, mm1, mv1, w_relation):
    raise NotImplementedError("write your pallas kernel here")


# speedup vs baseline: 5.6542x; 5.6542x over previous
import dataclasses
import functools

import jax
import jax.numpy as jnp
from jax import lax
from jax.experimental import pallas as pl
from jax.experimental.pallas import tpu as pltpu
from jax.experimental.pallas import tpu_sc as plsc

N = 10000
E = 320000
H = 128
R = 8
EPS = 1e-3

NC = 2
NS = 16
NW = NC * NS
EPT = E // NW
CH = 80
NCH = EPT // CH
CZ = 80
NZC = N // CZ
CC = 640
NCC = N * R // CC

f32 = jnp.float32
i32 = jnp.int32

_MESH = plsc.VectorSubcoreMesh(core_axis_name="c", subcore_axis_name="s")

_SC_PARAMS = pltpu.CompilerParams()
if "needs_layout_passes" in pltpu.CompilerParams.__dataclass_fields__:
    _SC_PARAMS = dataclasses.replace(_SC_PARAMS, needs_layout_passes=False)


_BLK = 2000


def _embed_body(x_ref, w_ref, b_ref, o_ref):
    o_ref[...] = jnp.dot(x_ref[...], w_ref[...],
                         preferred_element_type=f32) + b_ref[...]


def _embed(x, w, b):
    return pl.pallas_call(
        _embed_body,
        grid=(N // _BLK,),
        in_specs=[pl.BlockSpec((_BLK, H), lambda i: (i, 0)),
                  pl.BlockSpec((H, H), lambda i: (0, 0)),
                  pl.BlockSpec((1, H), lambda i: (0, 0))],
        out_specs=pl.BlockSpec((_BLK, H), lambda i: (i, 0)),
        out_shape=jax.ShapeDtypeStruct((N, H), f32),
    )(x, w, b)


def _proj_body(h_ref, w_ref, o_ref):
    o_ref[0] = jnp.dot(h_ref[...], w_ref[0], preferred_element_type=f32)


def _proj(h, wstack):
    return pl.pallas_call(
        _proj_body,
        grid=(R + 1, N // _BLK),
        in_specs=[pl.BlockSpec((_BLK, H), lambda r, i: (i, 0)),
                  pl.BlockSpec((1, H, H), lambda r, i: (r, 0, 0))],
        out_specs=pl.BlockSpec((1, _BLK, H), lambda r, i: (r, i, 0)),
        out_shape=jax.ShapeDtypeStruct((R + 1, N, H), f32),
    )(h, wstack)


def _bn_relu(v, mm, scale, bet):
    return jnp.maximum((v - mm) * scale + bet, 0.0)


def _post0_body(aggp_ref, hsl_ref, cnt_ref, gam_ref, bet_ref, mm_ref, mv_ref,
                h_ref, norm_ref):
    ctot = cnt_ref[0] + cnt_ref[1]
    norm = jnp.zeros((_BLK, 1), f32)
    for et in range(R):
        c = ctot[:, et:et + 1]
        norm = jnp.where(c > 0, 1.0 / jnp.maximum(c, 1.0), norm)
    v = (aggp_ref[0] + aggp_ref[1]) * norm + hsl_ref[...]
    scale = gam_ref[...] * lax.rsqrt(mv_ref[...] + EPS)
    h_ref[...] = _bn_relu(v, mm_ref[...], scale, bet_ref[...])
    norm_ref[...] = norm


def _post0(aggp, hsl, cnt, gam, bet, mm, mv):
    row = lambda i: (0, i, 0)
    vec = lambda i: (0, 0)
    return pl.pallas_call(
        _post0_body,
        grid=(N // _BLK,),
        in_specs=[pl.BlockSpec((NC, _BLK, H), row),
                  pl.BlockSpec((_BLK, H), lambda i: (i, 0)),
                  pl.BlockSpec((NC, _BLK, R), row),
                  pl.BlockSpec((1, H), vec),
                  pl.BlockSpec((1, H), vec),
                  pl.BlockSpec((1, H), vec),
                  pl.BlockSpec((1, H), vec)],
        out_specs=[pl.BlockSpec((_BLK, H), lambda i: (i, 0)),
                   pl.BlockSpec((_BLK, 1), lambda i: (i, 0))],
        out_shape=[jax.ShapeDtypeStruct((N, H), f32),
                   jax.ShapeDtypeStruct((N, 1), f32)],
    )(aggp, hsl, cnt, gam, bet, mm, mv)


def _post1_body(aggp_ref, hsl_ref, norm_ref, gam_ref, bet_ref, mm_ref, mv_ref,
                wrel_ref, tab_ref):
    v = (aggp_ref[0] + aggp_ref[1]) * norm_ref[...] + hsl_ref[...]
    scale = gam_ref[...] * lax.rsqrt(mv_ref[...] + EPS)
    h2 = _bn_relu(v, mm_ref[...], scale, bet_ref[...])
    for r in range(R):
        tab_ref[r] = h2 * wrel_ref[r]
    tab_ref[R] = h2


def _post1(aggp, hsl, norm, gam, bet, mm, mv, wrel):
    vec = lambda i: (0, 0)
    return pl.pallas_call(
        _post1_body,
        grid=(N // _BLK,),
        in_specs=[pl.BlockSpec((NC, _BLK, H), lambda i: (0, i, 0)),
                  pl.BlockSpec((_BLK, H), lambda i: (i, 0)),
                  pl.BlockSpec((_BLK, 1), lambda i: (i, 0)),
                  pl.BlockSpec((1, H), vec),
                  pl.BlockSpec((1, H), vec),
                  pl.BlockSpec((1, H), vec),
                  pl.BlockSpec((1, H), vec),
                  pl.BlockSpec((R, H), vec)],
        out_specs=pl.BlockSpec((R + 1, _BLK, H), lambda i: (0, i, 0)),
        out_shape=jax.ShapeDtypeStruct((R + 1, N, H), f32),
    )(aggp, hsl, norm, gam, bet, mm, mv, wrel)


def _zero_vmem_rows(buf, rows):
    zero16 = jnp.zeros((16,), f32)

    @pl.loop(0, rows)
    def _(r):
        @pl.loop(0, H // 16)
        def _(j):
            buf[r, pl.ds(j * 16, 16)] = zero16


def _make_agg(with_counts):
    out_type = [jax.ShapeDtypeStruct((NC, N, H), f32)]
    scratch = [
        pltpu.VMEM((CH,), i32),
        pltpu.VMEM((CH,), i32),
        pltpu.VMEM((CH,), i32),
        pltpu.VMEM((CH,), i32),
        pltpu.VMEM((CH, H), f32),
        pltpu.VMEM_SHARED((N, H), f32),
        pltpu.SemaphoreType.DMA,
    ]
    if with_counts:
        out_type.append(jax.ShapeDtypeStruct((NC, N * R), f32))
        scratch += [
            pltpu.VMEM((CH,), i32),
            pltpu.VMEM((CH,), f32),
            pltpu.VMEM((CC,), f32),
            pltpu.VMEM_SHARED((N * R,), f32),
        ]

    @functools.partial(pl.kernel, out_type=tuple(out_type), mesh=_MESH,
                       scratch_types=tuple(scratch))
    def agg(t_hbm, src_hbm, dst_hbm, et_hbm, agg_hbm, *rest):
        if with_counts:
            (cnt_hbm, srcv, dstv, etv, gidx, rows, agg_sh, sem,
             cidx, ones, z1d, cnt_sh) = rest
        else:
            (srcv, dstv, etv, gidx, rows, agg_sh, sem) = rest
        cid = lax.axis_index("c")
        sid = lax.axis_index("s")
        wid = cid * NS + sid

        _zero_vmem_rows(rows, CH)

        @pl.loop(sid, NZC, step=NS)
        def _(m):
            pltpu.sync_copy(rows, agg_sh.at[pl.ds(m * CZ, CZ)])
        if with_counts:
            zero16 = jnp.zeros((16,), f32)
            one16 = jnp.ones((16,), f32)

            @pl.loop(0, CC // 16)
            def _(j):
                z1d[pl.ds(j * 16, 16)] = zero16

            @pl.loop(0, CH // 16)
            def _(j):
                ones[pl.ds(j * 16, 16)] = one16

            @pl.loop(sid, NCC, step=NS)
            def _(m):
                pltpu.sync_copy(z1d, cnt_sh.at[pl.ds(m * CC, CC)])
        plsc.subcore_barrier()

        base = wid * EPT

        @pl.loop(0, NCH)
        def _(j):
            off = base + j * CH
            pltpu.sync_copy(src_hbm.at[pl.ds(off, CH)], srcv)
            pltpu.sync_copy(dst_hbm.at[pl.ds(off, CH)], dstv)
            pltpu.sync_copy(et_hbm.at[pl.ds(off, CH)], etv)

            @pl.loop(0, CH // 16)
            def _(k):
                sl = pl.ds(k * 16, 16)
                e16 = etv[sl]
                gidx[sl] = e16 * N + srcv[sl]
                if with_counts:
                    cidx[sl] = dstv[sl] * R + e16
            pltpu.async_copy(t_hbm.at[gidx], rows, sem).wait()
            pltpu.sync_copy(rows, agg_sh.at[dstv], add=True)
            if with_counts:
                pltpu.sync_copy(ones, cnt_sh.at[cidx], add=True)

        plsc.subcore_barrier()

        @pl.loop(sid, NZC, step=NS)
        def _(m):
            pltpu.sync_copy(agg_sh.at[pl.ds(m * CZ, CZ)],
                            agg_hbm.at[cid, pl.ds(m * CZ, CZ)])
        if with_counts:
            @pl.loop(sid, NCC, step=NS)
            def _(m):
                pltpu.sync_copy(cnt_sh.at[pl.ds(m * CC, CC)],
                                cnt_hbm.at[cid, pl.ds(m * CC, CC)])

    return agg


_agg_counts = _make_agg(True)
_agg_plain = _make_agg(False)


@functools.partial(
    pl.kernel,
    out_type=(jax.ShapeDtypeStruct((E,), f32), jax.ShapeDtypeStruct((E,), f32)),
    mesh=_MESH,
    scratch_types=(
        pltpu.VMEM((CH,), i32),
        pltpu.VMEM((CH,), i32),
        pltpu.VMEM((CH,), i32),
        pltpu.VMEM((CH,), i32),
        pltpu.VMEM((CH,), i32),
        pltpu.VMEM((CH,), i32),
        pltpu.VMEM((CH,), i32),
        pltpu.VMEM((CH, H), f32),
        pltpu.VMEM((CH, H), f32),
        pltpu.VMEM((CH, H), f32),
        pltpu.VMEM((CH,), f32),
        pltpu.VMEM((CH,), f32),
        pltpu.SemaphoreType.DMA,
    ),
    compiler_params=_SC_PARAMS,
)
def _score(tab_hbm, src_hbm, dst_hbm, et_hbm, nds_hbm, pos_hbm, neg_hbm,
           srcv, dstv, etv, ndsv, uidx, didx, nidx, urows, drows, nrows,
           posv, negv, sem):
    cid = lax.axis_index("c")
    sid = lax.axis_index("s")
    wid = cid * NS + sid
    base = wid * EPT

    @pl.loop(0, NCH)
    def _(j):
        off = base + j * CH
        pltpu.sync_copy(src_hbm.at[pl.ds(off, CH)], srcv)
        pltpu.sync_copy(dst_hbm.at[pl.ds(off, CH)], dstv)
        pltpu.sync_copy(et_hbm.at[pl.ds(off, CH)], etv)
        pltpu.sync_copy(nds_hbm.at[pl.ds(off, CH)], ndsv)

        @pl.loop(0, CH // 16)
        def _(k):
            sl = pl.ds(k * 16, 16)
            uidx[sl] = etv[sl] * N + srcv[sl]
            didx[sl] = dstv[sl] + R * N
            nidx[sl] = ndsv[sl] + R * N

        d1 = pltpu.async_copy(tab_hbm.at[uidx], urows, sem)
        d2 = pltpu.async_copy(tab_hbm.at[didx], drows, sem)
        d3 = pltpu.async_copy(tab_hbm.at[nidx], nrows, sem)
        d1.wait()
        d2.wait()
        d3.wait()

        for g in range(CH // 16):
            eids = lax.iota(i32, 16) + g * 16

            def cbody(c, carry):
                ap, an = carry
                col = jnp.full((16,), c, i32)
                u = plsc.load_gather(urows, [eids, col])
                dd = plsc.load_gather(drows, [eids, col])
                nn = plsc.load_gather(nrows, [eids, col])
                return ap + u * dd, an + u * nn

            ap, an = lax.fori_loop(0, H, cbody,
                                   (jnp.zeros((16,), f32),
                                    jnp.zeros((16,), f32)))
            posv[pl.ds(g * 16, 16)] = ap
            negv[pl.ds(g * 16, 16)] = an

        pltpu.sync_copy(posv, pos_hbm.at[pl.ds(off, CH)])
        pltpu.sync_copy(negv, neg_hbm.at[pl.ds(off, CH)])


def kernel(x, edge_index, edge_type, neg_dst, W_emb, b_emb,
           bases0, w_coe0, self_loop0, gamma0, beta0, mm0, mv0,
           bases1, w_coe1, self_loop1, gamma1, beta1, mm1, mv1,
           w_relation):
    src = edge_index[0]
    dst = edge_index[1]
    row = lambda a: a.reshape(1, H)

    w0 = jnp.concatenate(
        [jnp.einsum('ab,bcd->acd', w_coe0, bases0), self_loop0[None]], 0)
    w1 = jnp.concatenate(
        [jnp.einsum('ab,bcd->acd', w_coe1, bases1), self_loop1[None]], 0)

    h0 = _embed(x, W_emb, row(b_emb))
    t0 = _proj(h0, w0)
    aggp0, cnt = _agg_counts(t0.reshape((R + 1) * N, H), src, dst, edge_type)
    h1, norm = _post0(aggp0, t0[R], cnt.reshape(NC, N, R),
                      row(gamma0), row(beta0), row(mm0), row(mv0))
    t1 = _proj(h1, w1)
    (aggp1,) = _agg_plain(t1.reshape((R + 1) * N, H), src, dst, edge_type)
    tab = _post1(aggp1, t1[R], norm,
                 row(gamma1), row(beta1), row(mm1), row(mv1), w_relation)
    pos, neg = _score(tab.reshape((R + 1) * N, H),
                      src, dst, edge_type, neg_dst)
    return pos, neg


# speedup vs baseline: 6.9000x; 1.2203x over previous
import dataclasses
import functools

import jax
import jax.numpy as jnp
from jax import lax
from jax.experimental import pallas as pl
from jax.experimental.pallas import tpu as pltpu
from jax.experimental.pallas import tpu_sc as plsc

N = 10000
E = 320000
H = 128
R = 8
EPS = 1e-3

NC = 2
NS = 16
NW = NC * NS
EPT = E // NW
CH = 80
NCH = EPT // CH
CZ = 80
NZC = N // CZ
CC = 640
NCC = N * R // CC

f32 = jnp.float32
i32 = jnp.int32

_MESH = plsc.VectorSubcoreMesh(core_axis_name="c", subcore_axis_name="s")

_SC_PARAMS = pltpu.CompilerParams()
if "needs_layout_passes" in pltpu.CompilerParams.__dataclass_fields__:
    _SC_PARAMS = dataclasses.replace(_SC_PARAMS, needs_layout_passes=False)


_BLK = 2000


def _embed_body(x_ref, w_ref, b_ref, o_ref):
    o_ref[...] = jnp.dot(x_ref[...], w_ref[...],
                         preferred_element_type=f32) + b_ref[...]


def _embed(x, w, b):
    return pl.pallas_call(
        _embed_body,
        grid=(N // _BLK,),
        in_specs=[pl.BlockSpec((_BLK, H), lambda i: (i, 0)),
                  pl.BlockSpec((H, H), lambda i: (0, 0)),
                  pl.BlockSpec((1, H), lambda i: (0, 0))],
        out_specs=pl.BlockSpec((_BLK, H), lambda i: (i, 0)),
        out_shape=jax.ShapeDtypeStruct((N, H), f32),
    )(x, w, b)


def _proj_body(h_ref, w_ref, o_ref):
    o_ref[0] = jnp.dot(h_ref[...], w_ref[0], preferred_element_type=f32)


def _proj(h, wstack):
    return pl.pallas_call(
        _proj_body,
        grid=(R + 1, N // _BLK),
        in_specs=[pl.BlockSpec((_BLK, H), lambda r, i: (i, 0)),
                  pl.BlockSpec((1, H, H), lambda r, i: (r, 0, 0))],
        out_specs=pl.BlockSpec((1, _BLK, H), lambda r, i: (r, i, 0)),
        out_shape=jax.ShapeDtypeStruct((R + 1, N, H), f32),
    )(h, wstack)


def _bn_relu(v, mm, scale, bet):
    return jnp.maximum((v - mm) * scale + bet, 0.0)


def _post0_body(aggp_ref, hsl_ref, cnt_ref, gam_ref, bet_ref, mm_ref, mv_ref,
                h_ref, norm_ref):
    ctot = cnt_ref[0] + cnt_ref[1]
    norm = jnp.zeros((_BLK, 1), f32)
    for et in range(R):
        c = ctot[:, et:et + 1]
        norm = jnp.where(c > 0, 1.0 / jnp.maximum(c, 1.0), norm)
    v = (aggp_ref[0] + aggp_ref[1]) * norm + hsl_ref[...]
    scale = gam_ref[...] * lax.rsqrt(mv_ref[...] + EPS)
    h_ref[...] = _bn_relu(v, mm_ref[...], scale, bet_ref[...])
    norm_ref[...] = norm


def _post0(aggp, hsl, cnt, gam, bet, mm, mv):
    row = lambda i: (0, i, 0)
    vec = lambda i: (0, 0)
    return pl.pallas_call(
        _post0_body,
        grid=(N // _BLK,),
        in_specs=[pl.BlockSpec((NC, _BLK, H), row),
                  pl.BlockSpec((_BLK, H), lambda i: (i, 0)),
                  pl.BlockSpec((NC, _BLK, R), row),
                  pl.BlockSpec((1, H), vec),
                  pl.BlockSpec((1, H), vec),
                  pl.BlockSpec((1, H), vec),
                  pl.BlockSpec((1, H), vec)],
        out_specs=[pl.BlockSpec((_BLK, H), lambda i: (i, 0)),
                   pl.BlockSpec((_BLK, 1), lambda i: (i, 0))],
        out_shape=[jax.ShapeDtypeStruct((N, H), f32),
                   jax.ShapeDtypeStruct((N, 1), f32)],
    )(aggp, hsl, cnt, gam, bet, mm, mv)


def _post1_body(aggp_ref, hsl_ref, norm_ref, gam_ref, bet_ref, mm_ref, mv_ref,
                wrel_ref, tab_ref):
    v = (aggp_ref[0] + aggp_ref[1]) * norm_ref[...] + hsl_ref[...]
    scale = gam_ref[...] * lax.rsqrt(mv_ref[...] + EPS)
    h2 = _bn_relu(v, mm_ref[...], scale, bet_ref[...])
    for r in range(R):
        tab_ref[r] = h2 * wrel_ref[r]
    tab_ref[R] = h2


def _post1(aggp, hsl, norm, gam, bet, mm, mv, wrel):
    vec = lambda i: (0, 0)
    return pl.pallas_call(
        _post1_body,
        grid=(N // _BLK,),
        in_specs=[pl.BlockSpec((NC, _BLK, H), lambda i: (0, i, 0)),
                  pl.BlockSpec((_BLK, H), lambda i: (i, 0)),
                  pl.BlockSpec((_BLK, 1), lambda i: (i, 0)),
                  pl.BlockSpec((1, H), vec),
                  pl.BlockSpec((1, H), vec),
                  pl.BlockSpec((1, H), vec),
                  pl.BlockSpec((1, H), vec),
                  pl.BlockSpec((R, H), vec)],
        out_specs=pl.BlockSpec((R + 1, _BLK, H), lambda i: (0, i, 0)),
        out_shape=jax.ShapeDtypeStruct((R + 1, N, H), f32),
    )(aggp, hsl, norm, gam, bet, mm, mv, wrel)


def _zero_vmem_rows(buf, rows):
    zero16 = jnp.zeros((16,), f32)

    @pl.loop(0, rows)
    def _(r):
        @pl.loop(0, H // 16)
        def _(j):
            buf[r, pl.ds(j * 16, 16)] = zero16


def _make_agg(with_counts):
    out_type = [jax.ShapeDtypeStruct((NC, N, H), f32)]
    scratch = [
        pltpu.VMEM((CH,), i32),
        pltpu.VMEM((CH,), i32),
        pltpu.VMEM((CH,), i32),
        pltpu.VMEM((CH,), i32),
        pltpu.VMEM((CH, H), f32),
        pltpu.VMEM_SHARED((N, H), f32),
        pltpu.SemaphoreType.DMA,
    ]
    if with_counts:
        out_type.append(jax.ShapeDtypeStruct((NC, N * R), f32))
        scratch += [
            pltpu.VMEM((CH,), i32),
            pltpu.VMEM((CH,), f32),
            pltpu.VMEM((CC,), f32),
            pltpu.VMEM_SHARED((N * R,), f32),
        ]

    @functools.partial(pl.kernel, out_type=tuple(out_type), mesh=_MESH,
                       scratch_types=tuple(scratch))
    def agg(t_hbm, src_hbm, dst_hbm, et_hbm, agg_hbm, *rest):
        if with_counts:
            (cnt_hbm, srcv, dstv, etv, gidx, rows, agg_sh, sem,
             cidx, ones, z1d, cnt_sh) = rest
        else:
            (srcv, dstv, etv, gidx, rows, agg_sh, sem) = rest
        cid = lax.axis_index("c")
        sid = lax.axis_index("s")
        wid = cid * NS + sid

        _zero_vmem_rows(rows, CH)

        @pl.loop(sid, NZC, step=NS)
        def _(m):
            pltpu.sync_copy(rows, agg_sh.at[pl.ds(m * CZ, CZ)])
        if with_counts:
            zero16 = jnp.zeros((16,), f32)
            one16 = jnp.ones((16,), f32)

            @pl.loop(0, CC // 16)
            def _(j):
                z1d[pl.ds(j * 16, 16)] = zero16

            @pl.loop(0, CH // 16)
            def _(j):
                ones[pl.ds(j * 16, 16)] = one16

            @pl.loop(sid, NCC, step=NS)
            def _(m):
                pltpu.sync_copy(z1d, cnt_sh.at[pl.ds(m * CC, CC)])
        plsc.subcore_barrier()

        base = wid * EPT

        @pl.loop(0, NCH)
        def _(j):
            off = base + j * CH
            pltpu.sync_copy(src_hbm.at[pl.ds(off, CH)], srcv)
            pltpu.sync_copy(dst_hbm.at[pl.ds(off, CH)], dstv)
            pltpu.sync_copy(et_hbm.at[pl.ds(off, CH)], etv)

            @pl.loop(0, CH // 16)
            def _(k):
                sl = pl.ds(k * 16, 16)
                e16 = etv[sl]
                gidx[sl] = e16 * N + srcv[sl]
                if with_counts:
                    cidx[sl] = dstv[sl] * R + e16
            pltpu.async_copy(t_hbm.at[gidx], rows, sem).wait()
            pltpu.sync_copy(rows, agg_sh.at[dstv], add=True)
            if with_counts:
                pltpu.sync_copy(ones, cnt_sh.at[cidx], add=True)

        plsc.subcore_barrier()

        @pl.loop(sid, NZC, step=NS)
        def _(m):
            pltpu.sync_copy(agg_sh.at[pl.ds(m * CZ, CZ)],
                            agg_hbm.at[cid, pl.ds(m * CZ, CZ)])
        if with_counts:
            @pl.loop(sid, NCC, step=NS)
            def _(m):
                pltpu.sync_copy(cnt_sh.at[pl.ds(m * CC, CC)],
                                cnt_hbm.at[cid, pl.ds(m * CC, CC)])

    return agg


_agg_counts = _make_agg(True)
_agg_plain = _make_agg(False)


@functools.partial(
    pl.kernel,
    out_type=(jax.ShapeDtypeStruct((E,), f32), jax.ShapeDtypeStruct((E,), f32)),
    mesh=_MESH,
    scratch_types=(
        pltpu.VMEM((EPT,), i32),
        pltpu.VMEM((EPT,), i32),
        pltpu.VMEM((EPT,), i32),
        pltpu.VMEM((EPT,), i32),
        pltpu.VMEM((CH, H), f32),
        pltpu.VMEM((CH, H), f32),
        pltpu.VMEM((CH, H), f32),
        pltpu.VMEM((CH, H), f32),
        pltpu.VMEM((CH, H), f32),
        pltpu.VMEM((CH, H), f32),
        pltpu.VMEM((EPT,), f32),
        pltpu.VMEM((EPT,), f32),
        pltpu.SemaphoreType.DMA,
        pltpu.SemaphoreType.DMA,
    ),
    compiler_params=_SC_PARAMS,
)
def _score(tab_hbm, src_hbm, dst_hbm, et_hbm, nds_hbm, pos_hbm, neg_hbm,
           uidxf, didxf, etf, nidxf, urows0, urows1, drows0, drows1,
           nrows0, nrows1, posv, negv, sem0, sem1):
    cid = lax.axis_index("c")
    sid = lax.axis_index("s")
    wid = cid * NS + sid
    base = wid * EPT

    pltpu.sync_copy(src_hbm.at[pl.ds(base, EPT)], uidxf)
    pltpu.sync_copy(dst_hbm.at[pl.ds(base, EPT)], didxf)
    pltpu.sync_copy(et_hbm.at[pl.ds(base, EPT)], etf)
    pltpu.sync_copy(nds_hbm.at[pl.ds(base, EPT)], nidxf)

    @pl.loop(0, EPT // 16)
    def _(k):
        sl = pl.ds(k * 16, 16)
        uidxf[sl] = etf[sl] * N + uidxf[sl]
        didxf[sl] = didxf[sl] + R * N
        nidxf[sl] = nidxf[sl] + R * N

    def fire(j, ur, dr, nr, sem):
        sl = pl.ds(j * CH, CH)
        pltpu.async_copy(tab_hbm.at[uidxf.at[sl]], ur, sem)
        pltpu.async_copy(tab_hbm.at[didxf.at[sl]], dr, sem)
        pltpu.async_copy(tab_hbm.at[nidxf.at[sl]], nr, sem)

    def wait3(ur, dr, nr, sem):
        hsl = pl.ds(0, CH)
        pltpu.make_async_copy(tab_hbm.at[hsl], ur, sem).wait()
        pltpu.make_async_copy(tab_hbm.at[hsl], dr, sem).wait()
        pltpu.make_async_copy(tab_hbm.at[hsl], nr, sem).wait()

    def compute(j, ur, dr, nr):
        for g in range(CH // 16):
            eids = lax.iota(i32, 16) + g * 16

            def cbody(cb, carry):
                p0, p1, n0, n1 = carry
                for i in range(16):
                    col = jnp.full((16,), cb * 16 + i, i32)
                    u = plsc.load_gather(ur, [eids, col])
                    dd = plsc.load_gather(dr, [eids, col])
                    nn = plsc.load_gather(nr, [eids, col])
                    if i % 2 == 0:
                        p0 = p0 + u * dd
                        n0 = n0 + u * nn
                    else:
                        p1 = p1 + u * dd
                        n1 = n1 + u * nn
                return p0, p1, n0, n1

            z = jnp.zeros((16,), f32)
            p0, p1, n0, n1 = lax.fori_loop(0, H // 16, cbody, (z, z, z, z))
            posv[pl.ds(j * CH + g * 16, 16)] = p0 + p1
            negv[pl.ds(j * CH + g * 16, 16)] = n0 + n1

    fire(0, urows0, drows0, nrows0, sem0)

    @pl.loop(0, NCH + 1, step=2)
    def _(j):
        @pl.when(j + 1 < NCH)
        def _():
            fire(j + 1, urows1, drows1, nrows1, sem1)
        wait3(urows0, drows0, nrows0, sem0)
        compute(j, urows0, drows0, nrows0)

        @pl.when(j + 1 < NCH)
        def _():
            @pl.when(j + 2 < NCH)
            def _():
                fire(j + 2, urows0, drows0, nrows0, sem0)
            wait3(urows1, drows1, nrows1, sem1)
            compute(j + 1, urows1, drows1, nrows1)

    pltpu.sync_copy(posv, pos_hbm.at[pl.ds(base, EPT)])
    pltpu.sync_copy(negv, neg_hbm.at[pl.ds(base, EPT)])


def kernel(x, edge_index, edge_type, neg_dst, W_emb, b_emb,
           bases0, w_coe0, self_loop0, gamma0, beta0, mm0, mv0,
           bases1, w_coe1, self_loop1, gamma1, beta1, mm1, mv1,
           w_relation):
    src = edge_index[0]
    dst = edge_index[1]
    row = lambda a: a.reshape(1, H)

    w0 = jnp.concatenate(
        [jnp.einsum('ab,bcd->acd', w_coe0, bases0), self_loop0[None]], 0)
    w1 = jnp.concatenate(
        [jnp.einsum('ab,bcd->acd', w_coe1, bases1), self_loop1[None]], 0)

    h0 = _embed(x, W_emb, row(b_emb))
    t0 = _proj(h0, w0)
    aggp0, cnt = _agg_counts(t0.reshape((R + 1) * N, H), src, dst, edge_type)
    h1, norm = _post0(aggp0, t0[R], cnt.reshape(NC, N, R),
                      row(gamma0), row(beta0), row(mm0), row(mv0))
    t1 = _proj(h1, w1)
    (aggp1,) = _agg_plain(t1.reshape((R + 1) * N, H), src, dst, edge_type)
    tab = _post1(aggp1, t1[R], norm,
                 row(gamma1), row(beta1), row(mm1), row(mv1), w_relation)
    pos, neg = _score(tab.reshape((R + 1) * N, H),
                      src, dst, edge_type, neg_dst)
    return pos, neg


# speedup vs baseline: 14.5735x; 2.1121x over previous
import dataclasses
import functools

import jax
import jax.numpy as jnp
from jax import lax
from jax.experimental import pallas as pl
from jax.experimental.pallas import tpu as pltpu
from jax.experimental.pallas import tpu_sc as plsc

N = 10000
E = 320000
H = 128
R = 8
EPS = 1e-3

NC = 2
NS = 16
NW = NC * NS
EPT = E // NW
CH = 80
NCH = EPT // CH
CZ = 80
NZC = N // CZ
CC = 640
NCC = N * R // CC

f32 = jnp.float32
i32 = jnp.int32

_MESH = plsc.VectorSubcoreMesh(core_axis_name="c", subcore_axis_name="s")

_SC_PARAMS = pltpu.CompilerParams()
if "needs_layout_passes" in pltpu.CompilerParams.__dataclass_fields__:
    _SC_PARAMS = dataclasses.replace(_SC_PARAMS, needs_layout_passes=False)


_BLK = 2000


def _embed_body(x_ref, w_ref, b_ref, o_ref):
    o_ref[...] = jnp.dot(x_ref[...], w_ref[...],
                         preferred_element_type=f32) + b_ref[...]


def _embed(x, w, b):
    return pl.pallas_call(
        _embed_body,
        grid=(N // _BLK,),
        in_specs=[pl.BlockSpec((_BLK, H), lambda i: (i, 0)),
                  pl.BlockSpec((H, H), lambda i: (0, 0)),
                  pl.BlockSpec((1, H), lambda i: (0, 0))],
        out_specs=pl.BlockSpec((_BLK, H), lambda i: (i, 0)),
        out_shape=jax.ShapeDtypeStruct((N, H), f32),
    )(x, w, b)


def _proj_body(h_ref, w_ref, o_ref):
    o_ref[0] = jnp.dot(h_ref[...], w_ref[0], preferred_element_type=f32)


def _proj(h, wstack):
    return pl.pallas_call(
        _proj_body,
        grid=(R + 1, N // _BLK),
        in_specs=[pl.BlockSpec((_BLK, H), lambda r, i: (i, 0)),
                  pl.BlockSpec((1, H, H), lambda r, i: (r, 0, 0))],
        out_specs=pl.BlockSpec((1, _BLK, H), lambda r, i: (r, i, 0)),
        out_shape=jax.ShapeDtypeStruct((R + 1, N, H), f32),
    )(h, wstack)


def _bn_relu(v, mm, scale, bet):
    return jnp.maximum((v - mm) * scale + bet, 0.0)


def _post0_body(aggp_ref, hsl_ref, cnt_ref, gam_ref, bet_ref, mm_ref, mv_ref,
                h_ref, norm_ref):
    ctot = cnt_ref[0] + cnt_ref[1]
    norm = jnp.zeros((_BLK, 1), f32)
    for et in range(R):
        c = ctot[:, et:et + 1]
        norm = jnp.where(c > 0, 1.0 / jnp.maximum(c, 1.0), norm)
    v = (aggp_ref[0] + aggp_ref[1]) * norm + hsl_ref[...]
    scale = gam_ref[...] * lax.rsqrt(mv_ref[...] + EPS)
    h_ref[...] = _bn_relu(v, mm_ref[...], scale, bet_ref[...])
    norm_ref[...] = norm


def _post0(aggp, hsl, cnt, gam, bet, mm, mv):
    row = lambda i: (0, i, 0)
    vec = lambda i: (0, 0)
    return pl.pallas_call(
        _post0_body,
        grid=(N // _BLK,),
        in_specs=[pl.BlockSpec((NC, _BLK, H), row),
                  pl.BlockSpec((_BLK, H), lambda i: (i, 0)),
                  pl.BlockSpec((NC, _BLK, R), row),
                  pl.BlockSpec((1, H), vec),
                  pl.BlockSpec((1, H), vec),
                  pl.BlockSpec((1, H), vec),
                  pl.BlockSpec((1, H), vec)],
        out_specs=[pl.BlockSpec((_BLK, H), lambda i: (i, 0)),
                   pl.BlockSpec((_BLK, 1), lambda i: (i, 0))],
        out_shape=[jax.ShapeDtypeStruct((N, H), f32),
                   jax.ShapeDtypeStruct((N, 1), f32)],
    )(aggp, hsl, cnt, gam, bet, mm, mv)


def _post1_body(aggp_ref, hsl_ref, norm_ref, gam_ref, bet_ref, mm_ref, mv_ref,
                wrel_ref, tab_ref):
    v = (aggp_ref[0] + aggp_ref[1]) * norm_ref[...] + hsl_ref[...]
    scale = gam_ref[...] * lax.rsqrt(mv_ref[...] + EPS)
    h2 = _bn_relu(v, mm_ref[...], scale, bet_ref[...])
    for r in range(R):
        tab_ref[r] = h2 * wrel_ref[r]
    tab_ref[R] = h2


def _post1(aggp, hsl, norm, gam, bet, mm, mv, wrel):
    vec = lambda i: (0, 0)
    return pl.pallas_call(
        _post1_body,
        grid=(N // _BLK,),
        in_specs=[pl.BlockSpec((NC, _BLK, H), lambda i: (0, i, 0)),
                  pl.BlockSpec((_BLK, H), lambda i: (i, 0)),
                  pl.BlockSpec((_BLK, 1), lambda i: (i, 0)),
                  pl.BlockSpec((1, H), vec),
                  pl.BlockSpec((1, H), vec),
                  pl.BlockSpec((1, H), vec),
                  pl.BlockSpec((1, H), vec),
                  pl.BlockSpec((R, H), vec)],
        out_specs=pl.BlockSpec((R + 1, _BLK, H), lambda i: (0, i, 0)),
        out_shape=jax.ShapeDtypeStruct((R + 1, N, H), f32),
    )(aggp, hsl, norm, gam, bet, mm, mv, wrel)


def _zero_vmem_rows(buf, rows):
    zero16 = jnp.zeros((16,), f32)

    @pl.loop(0, rows)
    def _(r):
        @pl.loop(0, H // 16)
        def _(j):
            buf[r, pl.ds(j * 16, 16)] = zero16


def _make_agg(with_counts):
    out_type = [jax.ShapeDtypeStruct((NC, N, H), f32)]
    scratch = [
        pltpu.VMEM((CH,), i32),
        pltpu.VMEM((CH,), i32),
        pltpu.VMEM((CH,), i32),
        pltpu.VMEM((CH,), i32),
        pltpu.VMEM((CH, H), f32),
        pltpu.VMEM_SHARED((N, H), f32),
        pltpu.SemaphoreType.DMA,
    ]
    if with_counts:
        out_type.append(jax.ShapeDtypeStruct((NC, N * R), f32))
        scratch += [
            pltpu.VMEM((CH,), i32),
            pltpu.VMEM((CH,), f32),
            pltpu.VMEM((CC,), f32),
            pltpu.VMEM_SHARED((N * R,), f32),
        ]

    @functools.partial(pl.kernel, out_type=tuple(out_type), mesh=_MESH,
                       scratch_types=tuple(scratch))
    def agg(t_hbm, src_hbm, dst_hbm, et_hbm, agg_hbm, *rest):
        if with_counts:
            (cnt_hbm, srcv, dstv, etv, gidx, rows, agg_sh, sem,
             cidx, ones, z1d, cnt_sh) = rest
        else:
            (srcv, dstv, etv, gidx, rows, agg_sh, sem) = rest
        cid = lax.axis_index("c")
        sid = lax.axis_index("s")
        wid = cid * NS + sid

        _zero_vmem_rows(rows, CH)

        @pl.loop(sid, NZC, step=NS)
        def _(m):
            pltpu.sync_copy(rows, agg_sh.at[pl.ds(m * CZ, CZ)])
        if with_counts:
            zero16 = jnp.zeros((16,), f32)
            one16 = jnp.ones((16,), f32)

            @pl.loop(0, CC // 16)
            def _(j):
                z1d[pl.ds(j * 16, 16)] = zero16

            @pl.loop(0, CH // 16)
            def _(j):
                ones[pl.ds(j * 16, 16)] = one16

            @pl.loop(sid, NCC, step=NS)
            def _(m):
                pltpu.sync_copy(z1d, cnt_sh.at[pl.ds(m * CC, CC)])
        plsc.subcore_barrier()

        base = wid * EPT

        @pl.loop(0, NCH)
        def _(j):
            off = base + j * CH
            pltpu.sync_copy(src_hbm.at[pl.ds(off, CH)], srcv)
            pltpu.sync_copy(dst_hbm.at[pl.ds(off, CH)], dstv)
            pltpu.sync_copy(et_hbm.at[pl.ds(off, CH)], etv)

            @pl.loop(0, CH // 16)
            def _(k):
                sl = pl.ds(k * 16, 16)
                e16 = etv[sl]
                gidx[sl] = e16 * N + srcv[sl]
                if with_counts:
                    cidx[sl] = dstv[sl] * R + e16
            pltpu.async_copy(t_hbm.at[gidx], rows, sem).wait()
            pltpu.sync_copy(rows, agg_sh.at[dstv], add=True)
            if with_counts:
                pltpu.sync_copy(ones, cnt_sh.at[cidx], add=True)

        plsc.subcore_barrier()

        @pl.loop(sid, NZC, step=NS)
        def _(m):
            pltpu.sync_copy(agg_sh.at[pl.ds(m * CZ, CZ)],
                            agg_hbm.at[cid, pl.ds(m * CZ, CZ)])
        if with_counts:
            @pl.loop(sid, NCC, step=NS)
            def _(m):
                pltpu.sync_copy(cnt_sh.at[pl.ds(m * CC, CC)],
                                cnt_hbm.at[cid, pl.ds(m * CC, CC)])

    return agg


_agg_counts = _make_agg(True)
_agg_plain = _make_agg(False)


@functools.partial(
    pl.kernel,
    out_type=(jax.ShapeDtypeStruct((E,), f32), jax.ShapeDtypeStruct((E,), f32)),
    mesh=_MESH,
    scratch_types=(
        pltpu.VMEM((EPT,), i32),
        pltpu.VMEM((EPT,), i32),
        pltpu.VMEM((EPT,), i32),
        pltpu.VMEM((EPT,), i32),
        pltpu.VMEM((CH, H), f32),
        pltpu.VMEM((CH, H), f32),
        pltpu.VMEM((CH, H), f32),
        pltpu.VMEM((CH, H), f32),
        pltpu.VMEM((CH, H), f32),
        pltpu.VMEM((CH, H), f32),
        pltpu.VMEM((EPT,), f32),
        pltpu.VMEM((EPT,), f32),
        pltpu.VMEM((16, 16), f32),
        pltpu.VMEM((16, 16), f32),
        pltpu.SemaphoreType.DMA,
        pltpu.SemaphoreType.DMA,
    ),
    compiler_params=_SC_PARAMS,
)
def _score(tab_hbm, src_hbm, dst_hbm, et_hbm, nds_hbm, pos_hbm, neg_hbm,
           uidxf, didxf, etf, nidxf, urows0, urows1, drows0, drows1,
           nrows0, nrows1, posv, negv, tpbuf, tnbuf, sem0, sem1):
    cid = lax.axis_index("c")
    sid = lax.axis_index("s")
    wid = cid * NS + sid
    base = wid * EPT

    pltpu.sync_copy(src_hbm.at[pl.ds(base, EPT)], uidxf)
    pltpu.sync_copy(dst_hbm.at[pl.ds(base, EPT)], didxf)
    pltpu.sync_copy(et_hbm.at[pl.ds(base, EPT)], etf)
    pltpu.sync_copy(nds_hbm.at[pl.ds(base, EPT)], nidxf)

    @pl.loop(0, EPT // 16)
    def _(k):
        sl = pl.ds(k * 16, 16)
        uidxf[sl] = etf[sl] * N + uidxf[sl]
        didxf[sl] = didxf[sl] + R * N
        nidxf[sl] = nidxf[sl] + R * N

    def fire(j, ur, dr, nr, sem):
        sl = pl.ds(j * CH, CH)
        pltpu.async_copy(tab_hbm.at[uidxf.at[sl]], ur, sem)
        pltpu.async_copy(tab_hbm.at[didxf.at[sl]], dr, sem)
        pltpu.async_copy(tab_hbm.at[nidxf.at[sl]], nr, sem)

    def wait3(ur, dr, nr, sem):
        hsl = pl.ds(0, CH)
        pltpu.make_async_copy(tab_hbm.at[hsl], ur, sem).wait()
        pltpu.make_async_copy(tab_hbm.at[hsl], dr, sem).wait()
        pltpu.make_async_copy(tab_hbm.at[hsl], nr, sem).wait()

    def compute(j, ur, dr, nr):
        rows16 = lax.iota(i32, 16)

        @pl.loop(0, CH // 16)
        def _(g):
            for e in range(16):
                row = g * 16 + e
                p0 = p1 = n0 = n1 = jnp.zeros((16,), f32)
                for cb in range(H // 16):
                    sl = pl.ds(cb * 16, 16)
                    u = ur[row, sl]
                    dd = dr[row, sl]
                    nn = nr[row, sl]
                    if cb % 2 == 0:
                        p0 = p0 + u * dd
                        n0 = n0 + u * nn
                    else:
                        p1 = p1 + u * dd
                        n1 = n1 + u * nn
                tpbuf[e, pl.ds(0, 16)] = p0 + p1
                tnbuf[e, pl.ds(0, 16)] = n0 + n1
            ps0 = ps1 = ns0 = ns1 = jnp.zeros((16,), f32)
            for l in range(16):
                coll = jnp.full((16,), l, i32)
                if l % 2 == 0:
                    ps0 = ps0 + plsc.load_gather(tpbuf, [rows16, coll])
                    ns0 = ns0 + plsc.load_gather(tnbuf, [rows16, coll])
                else:
                    ps1 = ps1 + plsc.load_gather(tpbuf, [rows16, coll])
                    ns1 = ns1 + plsc.load_gather(tnbuf, [rows16, coll])
            posv[pl.ds(j * CH + g * 16, 16)] = ps0 + ps1
            negv[pl.ds(j * CH + g * 16, 16)] = ns0 + ns1

    fire(0, urows0, drows0, nrows0, sem0)

    @pl.loop(0, NCH + 1, step=2)
    def _(j):
        @pl.when(j + 1 < NCH)
        def _():
            fire(j + 1, urows1, drows1, nrows1, sem1)
        wait3(urows0, drows0, nrows0, sem0)
        compute(j, urows0, drows0, nrows0)

        @pl.when(j + 1 < NCH)
        def _():
            @pl.when(j + 2 < NCH)
            def _():
                fire(j + 2, urows0, drows0, nrows0, sem0)
            wait3(urows1, drows1, nrows1, sem1)
            compute(j + 1, urows1, drows1, nrows1)

    pltpu.sync_copy(posv, pos_hbm.at[pl.ds(base, EPT)])
    pltpu.sync_copy(negv, neg_hbm.at[pl.ds(base, EPT)])


def kernel(x, edge_index, edge_type, neg_dst, W_emb, b_emb,
           bases0, w_coe0, self_loop0, gamma0, beta0, mm0, mv0,
           bases1, w_coe1, self_loop1, gamma1, beta1, mm1, mv1,
           w_relation):
    src = edge_index[0]
    dst = edge_index[1]
    row = lambda a: a.reshape(1, H)

    w0 = jnp.concatenate(
        [jnp.einsum('ab,bcd->acd', w_coe0, bases0), self_loop0[None]], 0)
    w1 = jnp.concatenate(
        [jnp.einsum('ab,bcd->acd', w_coe1, bases1), self_loop1[None]], 0)

    h0 = _embed(x, W_emb, row(b_emb))
    t0 = _proj(h0, w0)
    aggp0, cnt = _agg_counts(t0.reshape((R + 1) * N, H), src, dst, edge_type)
    h1, norm = _post0(aggp0, t0[R], cnt.reshape(NC, N, R),
                      row(gamma0), row(beta0), row(mm0), row(mv0))
    t1 = _proj(h1, w1)
    (aggp1,) = _agg_plain(t1.reshape((R + 1) * N, H), src, dst, edge_type)
    tab = _post1(aggp1, t1[R], norm,
                 row(gamma1), row(beta1), row(mm1), row(mv1), w_relation)
    pos, neg = _score(tab.reshape((R + 1) * N, H),
                      src, dst, edge_type, neg_dst)
    return pos, neg


# speedup vs baseline: 24.7771x; 1.7002x over previous
import dataclasses
import functools

import jax
import jax.numpy as jnp
from jax import lax
from jax.experimental import pallas as pl
from jax.experimental.pallas import tpu as pltpu
from jax.experimental.pallas import tpu_sc as plsc

N = 10000
E = 320000
H = 128
R = 8
EPS = 1e-3

NC = 2
NS = 16
NW = NC * NS
EPT = E // NW
CH = 80
NCH = EPT // CH
CZ = 80
NZC = N // CZ
CC = 640
NCC = N * R // CC

f32 = jnp.float32
i32 = jnp.int32

_MESH = plsc.VectorSubcoreMesh(core_axis_name="c", subcore_axis_name="s")

_SC_PARAMS = pltpu.CompilerParams()
if "needs_layout_passes" in pltpu.CompilerParams.__dataclass_fields__:
    _SC_PARAMS = dataclasses.replace(_SC_PARAMS, needs_layout_passes=False)


_BLK = 2000


def _embed_body(x_ref, w_ref, b_ref, o_ref):
    o_ref[...] = jnp.dot(x_ref[...], w_ref[...],
                         preferred_element_type=f32) + b_ref[...]


def _embed(x, w, b):
    return pl.pallas_call(
        _embed_body,
        grid=(N // _BLK,),
        in_specs=[pl.BlockSpec((_BLK, H), lambda i: (i, 0)),
                  pl.BlockSpec((H, H), lambda i: (0, 0)),
                  pl.BlockSpec((1, H), lambda i: (0, 0))],
        out_specs=pl.BlockSpec((_BLK, H), lambda i: (i, 0)),
        out_shape=jax.ShapeDtypeStruct((N, H), f32),
    )(x, w, b)


def _proj_body(h_ref, w_ref, o_ref):
    o_ref[0] = jnp.dot(h_ref[...], w_ref[0], preferred_element_type=f32)


def _proj(h, wstack):
    return pl.pallas_call(
        _proj_body,
        grid=(R + 1, N // _BLK),
        in_specs=[pl.BlockSpec((_BLK, H), lambda r, i: (i, 0)),
                  pl.BlockSpec((1, H, H), lambda r, i: (r, 0, 0))],
        out_specs=pl.BlockSpec((1, _BLK, H), lambda r, i: (r, i, 0)),
        out_shape=jax.ShapeDtypeStruct((R + 1, N, H), f32),
    )(h, wstack)


def _bn_relu(v, mm, scale, bet):
    return jnp.maximum((v - mm) * scale + bet, 0.0)


def _post0_body(aggp_ref, hsl_ref, cnt_ref, gam_ref, bet_ref, mm_ref, mv_ref,
                h_ref, norm_ref):
    ctot = cnt_ref[0] + cnt_ref[1]
    norm = jnp.zeros((_BLK, 1), f32)
    for et in range(R):
        c = ctot[:, et:et + 1]
        norm = jnp.where(c > 0, 1.0 / jnp.maximum(c, 1.0), norm)
    v = (aggp_ref[0] + aggp_ref[1]) * norm + hsl_ref[...]
    scale = gam_ref[...] * lax.rsqrt(mv_ref[...] + EPS)
    h_ref[...] = _bn_relu(v, mm_ref[...], scale, bet_ref[...])
    norm_ref[...] = norm


def _post0(aggp, hsl, cnt, gam, bet, mm, mv):
    row = lambda i: (0, i, 0)
    vec = lambda i: (0, 0)
    return pl.pallas_call(
        _post0_body,
        grid=(N // _BLK,),
        in_specs=[pl.BlockSpec((NC, _BLK, H), row),
                  pl.BlockSpec((_BLK, H), lambda i: (i, 0)),
                  pl.BlockSpec((NC, _BLK, R), row),
                  pl.BlockSpec((1, H), vec),
                  pl.BlockSpec((1, H), vec),
                  pl.BlockSpec((1, H), vec),
                  pl.BlockSpec((1, H), vec)],
        out_specs=[pl.BlockSpec((_BLK, H), lambda i: (i, 0)),
                   pl.BlockSpec((_BLK, 1), lambda i: (i, 0))],
        out_shape=[jax.ShapeDtypeStruct((N, H), f32),
                   jax.ShapeDtypeStruct((N, 1), f32)],
    )(aggp, hsl, cnt, gam, bet, mm, mv)


def _post1_body(aggp_ref, hsl_ref, norm_ref, gam_ref, bet_ref, mm_ref, mv_ref,
                wrel_ref, tab_ref):
    v = (aggp_ref[0] + aggp_ref[1]) * norm_ref[...] + hsl_ref[...]
    scale = gam_ref[...] * lax.rsqrt(mv_ref[...] + EPS)
    h2 = _bn_relu(v, mm_ref[...], scale, bet_ref[...])
    for r in range(R):
        tab_ref[r] = h2 * wrel_ref[r]
    tab_ref[R] = h2


def _post1(aggp, hsl, norm, gam, bet, mm, mv, wrel):
    vec = lambda i: (0, 0)
    return pl.pallas_call(
        _post1_body,
        grid=(N // _BLK,),
        in_specs=[pl.BlockSpec((NC, _BLK, H), lambda i: (0, i, 0)),
                  pl.BlockSpec((_BLK, H), lambda i: (i, 0)),
                  pl.BlockSpec((_BLK, 1), lambda i: (i, 0)),
                  pl.BlockSpec((1, H), vec),
                  pl.BlockSpec((1, H), vec),
                  pl.BlockSpec((1, H), vec),
                  pl.BlockSpec((1, H), vec),
                  pl.BlockSpec((R, H), vec)],
        out_specs=pl.BlockSpec((R + 1, _BLK, H), lambda i: (0, i, 0)),
        out_shape=jax.ShapeDtypeStruct((R + 1, N, H), f32),
    )(aggp, hsl, norm, gam, bet, mm, mv, wrel)


def _zero_vmem_rows(buf, rows):
    zero16 = jnp.zeros((16,), f32)

    @pl.loop(0, rows)
    def _(r):
        @pl.loop(0, H // 16)
        def _(j):
            buf[r, pl.ds(j * 16, 16)] = zero16


def _make_agg(with_counts):
    out_type = [jax.ShapeDtypeStruct((NC, N, H), f32)]
    scratch = [
        pltpu.VMEM((3, CH), i32),
        pltpu.VMEM((3, CH), i32),
        pltpu.VMEM((3, CH), i32),
        pltpu.VMEM((CH, H), f32),
        pltpu.VMEM((CH, H), f32),
        pltpu.VMEM((CH, H), f32),
        pltpu.VMEM((CH,), i32),
        pltpu.VMEM((CH,), i32),
        pltpu.VMEM((CH,), i32),
        pltpu.VMEM((CH,), i32),
        pltpu.VMEM((CH,), i32),
        pltpu.VMEM((CH,), i32),
        pltpu.VMEM_SHARED((N, H), f32),
        pltpu.SemaphoreType.DMA,
        pltpu.SemaphoreType.DMA,
        pltpu.SemaphoreType.DMA,
        pltpu.SemaphoreType.DMA,
        pltpu.SemaphoreType.DMA,
        pltpu.SemaphoreType.DMA,
        pltpu.SemaphoreType.DMA,
        pltpu.SemaphoreType.DMA,
        pltpu.SemaphoreType.DMA,
    ]
    if with_counts:
        out_type.append(jax.ShapeDtypeStruct((NC, N * R), f32))
        scratch += [
            pltpu.VMEM((CH,), i32),
            pltpu.VMEM((CH,), i32),
            pltpu.VMEM((CH,), i32),
            pltpu.VMEM((CH,), f32),
            pltpu.VMEM((CC,), f32),
            pltpu.VMEM_SHARED((N * R,), f32),
            pltpu.SemaphoreType.DMA,
            pltpu.SemaphoreType.DMA,
            pltpu.SemaphoreType.DMA,
        ]

    @functools.partial(pl.kernel, out_type=tuple(out_type), mesh=_MESH,
                       scratch_types=tuple(scratch))
    def agg(t_hbm, src_hbm, dst_hbm, et_hbm, agg_hbm, *rest):
        if with_counts:
            (cnt_hbm, srcb, dstb, etb, r0, r1, r2, g0, g1, g2,
             dc0, dc1, dc2, agg_sh, si0, si1, si2,
             sg0, sg1, sg2, sa0, sa1, sa2,
             ci0, ci1, ci2, ones, z1d, cnt_sh, sc0, sc1, sc2) = rest
            cidc = (ci0, ci1, ci2)
            semc = (sc0, sc1, sc2)
        else:
            (srcb, dstb, etb, r0, r1, r2, g0, g1, g2,
             dc0, dc1, dc2, agg_sh, si0, si1, si2,
             sg0, sg1, sg2, sa0, sa1, sa2) = rest
            cidc = semc = None
        rows = (r0, r1, r2)
        gidx = (g0, g1, g2)
        dstc = (dc0, dc1, dc2)
        semi = (si0, si1, si2)
        semg = (sg0, sg1, sg2)
        sema = (sa0, sa1, sa2)
        cid = lax.axis_index("c")
        sid = lax.axis_index("s")
        wid = cid * NS + sid
        base = wid * EPT

        _zero_vmem_rows(r0, CH)

        @pl.loop(sid, NZC, step=NS)
        def _(m):
            pltpu.sync_copy(r0, agg_sh.at[pl.ds(m * CZ, CZ)])
        if with_counts:
            zero16 = jnp.zeros((16,), f32)
            one16 = jnp.ones((16,), f32)

            @pl.loop(0, CC // 16)
            def _(j):
                z1d[pl.ds(j * 16, 16)] = zero16

            @pl.loop(0, CH // 16)
            def _(j):
                ones[pl.ds(j * 16, 16)] = one16

            @pl.loop(sid, NCC, step=NS)
            def _(m):
                pltpu.sync_copy(z1d, cnt_sh.at[pl.ds(m * CC, CC)])
        plsc.subcore_barrier()

        def fire_idx(j, b):
            sl = pl.ds(base + j * CH, CH)
            pltpu.async_copy(src_hbm.at[sl], srcb.at[b], semi[b])
            pltpu.async_copy(dst_hbm.at[sl], dstb.at[b], semi[b])
            pltpu.async_copy(et_hbm.at[sl], etb.at[b], semi[b])

        def wait_idx(b):
            hsl = pl.ds(0, CH)
            pltpu.make_async_copy(src_hbm.at[hsl], srcb.at[b],
                                  semi[b]).wait()
            pltpu.make_async_copy(src_hbm.at[hsl], dstb.at[b],
                                  semi[b]).wait()
            pltpu.make_async_copy(src_hbm.at[hsl], etb.at[b],
                                  semi[b]).wait()

        def build_idx(b):
            @pl.loop(0, CH // 16)
            def _(k):
                sl = pl.ds(k * 16, 16)
                e16 = etb[b, sl]
                gidx[b][sl] = e16 * N + srcb[b, sl]
                dstc[b][sl] = dstb[b, sl]
                if with_counts:
                    cidc[b][sl] = dstb[b, sl] * R + e16

        def fire_gather(j, b):
            pltpu.async_copy(t_hbm.at[gidx[b]], rows[b], semg[b])

        def wait_gather(b):
            pltpu.make_async_copy(t_hbm.at[pl.ds(0, CH)], rows[b],
                                  semg[b]).wait()

        def wait_scatter(b):
            pltpu.make_async_copy(rows[b], agg_sh.at[pl.ds(0, CH)],
                                  sema[b]).wait()
            if with_counts:
                pltpu.make_async_copy(ones, cnt_sh.at[pl.ds(0, CH)],
                                      semc[b]).wait()

        def half(jj, b):
            @pl.when(jj < NCH)
            def _():
                nb = (b + 1) % 3

                @pl.when(jj + 2 < NCH)
                def _():
                    fire_idx(jj + 2, (b + 2) % 3)

                @pl.when(jj + 1 < NCH)
                def _():
                    @pl.when(jj >= 2)
                    def _():
                        wait_scatter(nb)
                    wait_idx(nb)
                    build_idx(nb)
                    fire_gather(jj + 1, nb)
                wait_gather(b)
                pltpu.async_copy(rows[b], agg_sh.at[dstc[b]], sema[b],
                                 add=True)
                if with_counts:
                    pltpu.async_copy(ones, cnt_sh.at[cidc[b]], semc[b],
                                     add=True)

        fire_idx(0, 0)
        fire_idx(1, 1)
        wait_idx(0)
        build_idx(0)
        fire_gather(0, 0)

        @pl.loop(0, NCH + 2, step=3)
        def _(j):
            half(j, 0)
            half(j + 1, 1)
            half(j + 2, 2)

        wait_scatter((NCH - 3) % 3)
        wait_scatter((NCH - 2) % 3)
        wait_scatter((NCH - 1) % 3)
        plsc.subcore_barrier()

        @pl.loop(sid, NZC, step=NS)
        def _(m):
            pltpu.sync_copy(agg_sh.at[pl.ds(m * CZ, CZ)],
                            agg_hbm.at[cid, pl.ds(m * CZ, CZ)])
        if with_counts:
            @pl.loop(sid, NCC, step=NS)
            def _(m):
                pltpu.sync_copy(cnt_sh.at[pl.ds(m * CC, CC)],
                                cnt_hbm.at[cid, pl.ds(m * CC, CC)])

    return agg


_agg_counts = _make_agg(True)
_agg_plain = _make_agg(False)


@functools.partial(
    pl.kernel,
    out_type=(jax.ShapeDtypeStruct((E,), f32), jax.ShapeDtypeStruct((E,), f32)),
    mesh=_MESH,
    scratch_types=(
        pltpu.VMEM((EPT,), i32),
        pltpu.VMEM((EPT,), i32),
        pltpu.VMEM((EPT,), i32),
        pltpu.VMEM((EPT,), i32),
        pltpu.VMEM((CH, H), f32),
        pltpu.VMEM((CH, H), f32),
        pltpu.VMEM((CH, H), f32),
        pltpu.VMEM((CH, H), f32),
        pltpu.VMEM((CH, H), f32),
        pltpu.VMEM((CH, H), f32),
        pltpu.VMEM((EPT,), f32),
        pltpu.VMEM((EPT,), f32),
        pltpu.VMEM((16, 16), f32),
        pltpu.VMEM((16, 16), f32),
        pltpu.SemaphoreType.DMA,
        pltpu.SemaphoreType.DMA,
    ),
    compiler_params=_SC_PARAMS,
)
def _score(tab_hbm, src_hbm, dst_hbm, et_hbm, nds_hbm, pos_hbm, neg_hbm,
           uidxf, didxf, etf, nidxf, urows0, urows1, drows0, drows1,
           nrows0, nrows1, posv, negv, tpbuf, tnbuf, sem0, sem1):
    cid = lax.axis_index("c")
    sid = lax.axis_index("s")
    wid = cid * NS + sid
    base = wid * EPT

    pltpu.sync_copy(src_hbm.at[pl.ds(base, EPT)], uidxf)
    pltpu.sync_copy(dst_hbm.at[pl.ds(base, EPT)], didxf)
    pltpu.sync_copy(et_hbm.at[pl.ds(base, EPT)], etf)
    pltpu.sync_copy(nds_hbm.at[pl.ds(base, EPT)], nidxf)

    @pl.loop(0, EPT // 16)
    def _(k):
        sl = pl.ds(k * 16, 16)
        uidxf[sl] = etf[sl] * N + uidxf[sl]
        didxf[sl] = didxf[sl] + R * N
        nidxf[sl] = nidxf[sl] + R * N

    def fire(j, ur, dr, nr, sem):
        sl = pl.ds(j * CH, CH)
        pltpu.async_copy(tab_hbm.at[uidxf.at[sl]], ur, sem)
        pltpu.async_copy(tab_hbm.at[didxf.at[sl]], dr, sem)
        pltpu.async_copy(tab_hbm.at[nidxf.at[sl]], nr, sem)

    def wait3(ur, dr, nr, sem):
        hsl = pl.ds(0, CH)
        pltpu.make_async_copy(tab_hbm.at[hsl], ur, sem).wait()
        pltpu.make_async_copy(tab_hbm.at[hsl], dr, sem).wait()
        pltpu.make_async_copy(tab_hbm.at[hsl], nr, sem).wait()

    def compute(j, ur, dr, nr):
        rows16 = lax.iota(i32, 16)

        @pl.loop(0, CH // 16)
        def _(g):
            for e in range(16):
                row = g * 16 + e
                p0 = p1 = n0 = n1 = jnp.zeros((16,), f32)
                for cb in range(H // 16):
                    sl = pl.ds(cb * 16, 16)
                    u = ur[row, sl]
                    dd = dr[row, sl]
                    nn = nr[row, sl]
                    if cb % 2 == 0:
                        p0 = p0 + u * dd
                        n0 = n0 + u * nn
                    else:
                        p1 = p1 + u * dd
                        n1 = n1 + u * nn
                tpbuf[e, pl.ds(0, 16)] = p0 + p1
                tnbuf[e, pl.ds(0, 16)] = n0 + n1
            ps0 = ps1 = ns0 = ns1 = jnp.zeros((16,), f32)
            for l in range(16):
                coll = jnp.full((16,), l, i32)
                if l % 2 == 0:
                    ps0 = ps0 + plsc.load_gather(tpbuf, [rows16, coll])
                    ns0 = ns0 + plsc.load_gather(tnbuf, [rows16, coll])
                else:
                    ps1 = ps1 + plsc.load_gather(tpbuf, [rows16, coll])
                    ns1 = ns1 + plsc.load_gather(tnbuf, [rows16, coll])
            posv[pl.ds(j * CH + g * 16, 16)] = ps0 + ps1
            negv[pl.ds(j * CH + g * 16, 16)] = ns0 + ns1

    fire(0, urows0, drows0, nrows0, sem0)

    @pl.loop(0, NCH + 1, step=2)
    def _(j):
        @pl.when(j + 1 < NCH)
        def _():
            fire(j + 1, urows1, drows1, nrows1, sem1)
        wait3(urows0, drows0, nrows0, sem0)
        compute(j, urows0, drows0, nrows0)

        @pl.when(j + 1 < NCH)
        def _():
            @pl.when(j + 2 < NCH)
            def _():
                fire(j + 2, urows0, drows0, nrows0, sem0)
            wait3(urows1, drows1, nrows1, sem1)
            compute(j + 1, urows1, drows1, nrows1)

    pltpu.sync_copy(posv, pos_hbm.at[pl.ds(base, EPT)])
    pltpu.sync_copy(negv, neg_hbm.at[pl.ds(base, EPT)])


def kernel(x, edge_index, edge_type, neg_dst, W_emb, b_emb,
           bases0, w_coe0, self_loop0, gamma0, beta0, mm0, mv0,
           bases1, w_coe1, self_loop1, gamma1, beta1, mm1, mv1,
           w_relation):
    src = edge_index[0]
    dst = edge_index[1]
    row = lambda a: a.reshape(1, H)

    w0 = jnp.concatenate(
        [jnp.einsum('ab,bcd->acd', w_coe0, bases0), self_loop0[None]], 0)
    w1 = jnp.concatenate(
        [jnp.einsum('ab,bcd->acd', w_coe1, bases1), self_loop1[None]], 0)

    h0 = _embed(x, W_emb, row(b_emb))
    t0 = _proj(h0, w0)
    aggp0, cnt = _agg_counts(t0.reshape((R + 1) * N, H), src, dst, edge_type)
    h1, norm = _post0(aggp0, t0[R], cnt.reshape(NC, N, R),
                      row(gamma0), row(beta0), row(mm0), row(mv0))
    t1 = _proj(h1, w1)
    (aggp1,) = _agg_plain(t1.reshape((R + 1) * N, H), src, dst, edge_type)
    tab = _post1(aggp1, t1[R], norm,
                 row(gamma1), row(beta1), row(mm1), row(mv1), w_relation)
    pos, neg = _score(tab.reshape((R + 1) * N, H),
                      src, dst, edge_type, neg_dst)
    return pos, neg


# speedup vs baseline: 26.2961x; 1.0613x over previous
import dataclasses
import functools

import jax
import jax.numpy as jnp
from jax import lax
from jax.experimental import pallas as pl
from jax.experimental.pallas import tpu as pltpu
from jax.experimental.pallas import tpu_sc as plsc

N = 10000
E = 320000
H = 128
R = 8
EPS = 1e-3

NC = 2
NS = 16
NW = NC * NS
EPT = E // NW
CH = 80
NCH = EPT // CH
CZ = 80
NZC = N // CZ
CC = 640
NCC = N * R // CC

f32 = jnp.float32
i32 = jnp.int32

_MESH = plsc.VectorSubcoreMesh(core_axis_name="c", subcore_axis_name="s")

_SC_PARAMS = pltpu.CompilerParams()
if "needs_layout_passes" in pltpu.CompilerParams.__dataclass_fields__:
    _SC_PARAMS = dataclasses.replace(_SC_PARAMS, needs_layout_passes=False)


_BLK = 2000


def _proj_body(x_ref, w_ref, b_ref, o_ref):
    o_ref[0] = jnp.dot(x_ref[...], w_ref[0],
                       preferred_element_type=f32) + b_ref[0]


def _proj(x, wstack, bstack):
    return pl.pallas_call(
        _proj_body,
        grid=(R + 1, N // _BLK),
        in_specs=[pl.BlockSpec((_BLK, H), lambda r, i: (i, 0)),
                  pl.BlockSpec((1, H, H), lambda r, i: (r, 0, 0)),
                  pl.BlockSpec((1, 1, H), lambda r, i: (r, 0, 0))],
        out_specs=pl.BlockSpec((1, _BLK, H), lambda r, i: (r, i, 0)),
        out_shape=jax.ShapeDtypeStruct((R + 1, N, H), f32),
    )(x, wstack, bstack)


def _bn_relu(v, mm, scale, bet):
    return jnp.maximum((v - mm) * scale + bet, 0.0)


def _postproj_body(aggp_ref, hsl_ref, cnt_ref, gam_ref, bet_ref, mm_ref,
                   mv_ref, w1_ref, t_ref, norm_ref):
    ctot = cnt_ref[0] + cnt_ref[1]
    norm = jnp.zeros((_BLK, 1), f32)
    for et in range(R):
        c = ctot[:, et:et + 1]
        norm = jnp.where(c > 0, 1.0 / jnp.maximum(c, 1.0), norm)
    v = (aggp_ref[0] + aggp_ref[1]) * norm + hsl_ref[...]
    scale = gam_ref[...] * lax.rsqrt(mv_ref[...] + EPS)
    h1 = _bn_relu(v, mm_ref[...], scale, bet_ref[...])
    for r in range(R + 1):
        t_ref[r] = jnp.dot(h1, w1_ref[r], preferred_element_type=f32)
    norm_ref[...] = norm


def _postproj(aggp, hsl, cnt, gam, bet, mm, mv, w1):
    row = lambda i: (0, i, 0)
    vec = lambda i: (0, 0)
    return pl.pallas_call(
        _postproj_body,
        grid=(N // _BLK,),
        in_specs=[pl.BlockSpec((NC, _BLK, H), row),
                  pl.BlockSpec((_BLK, H), lambda i: (i, 0)),
                  pl.BlockSpec((NC, _BLK, R), row),
                  pl.BlockSpec((1, H), vec),
                  pl.BlockSpec((1, H), vec),
                  pl.BlockSpec((1, H), vec),
                  pl.BlockSpec((1, H), vec),
                  pl.BlockSpec((R + 1, H, H), lambda i: (0, 0, 0))],
        out_specs=[pl.BlockSpec((R + 1, _BLK, H), lambda i: (0, i, 0)),
                   pl.BlockSpec((_BLK, 1), lambda i: (i, 0))],
        out_shape=[jax.ShapeDtypeStruct((R + 1, N, H), f32),
                   jax.ShapeDtypeStruct((N, 1), f32)],
    )(aggp, hsl, cnt, gam, bet, mm, mv, w1)


def _post1_body(aggp_ref, hsl_ref, norm_ref, gam_ref, bet_ref, mm_ref, mv_ref,
                wrel_ref, tab_ref):
    v = (aggp_ref[0] + aggp_ref[1]) * norm_ref[...] + hsl_ref[...]
    scale = gam_ref[...] * lax.rsqrt(mv_ref[...] + EPS)
    h2 = _bn_relu(v, mm_ref[...], scale, bet_ref[...])
    for r in range(R):
        tab_ref[r] = h2 * wrel_ref[r]
    tab_ref[R] = h2


def _post1(aggp, hsl, norm, gam, bet, mm, mv, wrel):
    vec = lambda i: (0, 0)
    return pl.pallas_call(
        _post1_body,
        grid=(N // _BLK,),
        in_specs=[pl.BlockSpec((NC, _BLK, H), lambda i: (0, i, 0)),
                  pl.BlockSpec((_BLK, H), lambda i: (i, 0)),
                  pl.BlockSpec((_BLK, 1), lambda i: (i, 0)),
                  pl.BlockSpec((1, H), vec),
                  pl.BlockSpec((1, H), vec),
                  pl.BlockSpec((1, H), vec),
                  pl.BlockSpec((1, H), vec),
                  pl.BlockSpec((R, H), vec)],
        out_specs=pl.BlockSpec((R + 1, _BLK, H), lambda i: (0, i, 0)),
        out_shape=jax.ShapeDtypeStruct((R + 1, N, H), f32),
    )(aggp, hsl, norm, gam, bet, mm, mv, wrel)


def _zero_vmem_rows(buf, rows):
    zero16 = jnp.zeros((16,), f32)

    @pl.loop(0, rows)
    def _(r):
        @pl.loop(0, H // 16)
        def _(j):
            buf[r, pl.ds(j * 16, 16)] = zero16


def _make_agg(with_counts):
    out_type = [jax.ShapeDtypeStruct((NC, N, H), f32)]
    scratch = [
        pltpu.VMEM((3, CH), i32),
        pltpu.VMEM((3, CH), i32),
        pltpu.VMEM((3, CH), i32),
        pltpu.VMEM((CH, H), f32),
        pltpu.VMEM((CH, H), f32),
        pltpu.VMEM((CH, H), f32),
        pltpu.VMEM((CH,), i32),
        pltpu.VMEM((CH,), i32),
        pltpu.VMEM((CH,), i32),
        pltpu.VMEM((CH,), i32),
        pltpu.VMEM((CH,), i32),
        pltpu.VMEM((CH,), i32),
        pltpu.VMEM_SHARED((N, H), f32),
        pltpu.SemaphoreType.DMA,
        pltpu.SemaphoreType.DMA,
        pltpu.SemaphoreType.DMA,
        pltpu.SemaphoreType.DMA,
        pltpu.SemaphoreType.DMA,
        pltpu.SemaphoreType.DMA,
        pltpu.SemaphoreType.DMA,
        pltpu.SemaphoreType.DMA,
        pltpu.SemaphoreType.DMA,
    ]
    if with_counts:
        out_type.append(jax.ShapeDtypeStruct((NC, N * R), f32))
        scratch += [
            pltpu.VMEM((CH,), i32),
            pltpu.VMEM((CH,), i32),
            pltpu.VMEM((CH,), i32),
            pltpu.VMEM((CH,), f32),
            pltpu.VMEM((CC,), f32),
            pltpu.VMEM_SHARED((N * R,), f32),
            pltpu.SemaphoreType.DMA,
            pltpu.SemaphoreType.DMA,
            pltpu.SemaphoreType.DMA,
        ]

    @functools.partial(pl.kernel, out_type=tuple(out_type), mesh=_MESH,
                       scratch_types=tuple(scratch))
    def agg(t_hbm, src_hbm, dst_hbm, et_hbm, agg_hbm, *rest):
        if with_counts:
            (cnt_hbm, srcb, dstb, etb, r0, r1, r2, g0, g1, g2,
             dc0, dc1, dc2, agg_sh, si0, si1, si2,
             sg0, sg1, sg2, sa0, sa1, sa2,
             ci0, ci1, ci2, ones, z1d, cnt_sh, sc0, sc1, sc2) = rest
            cidc = (ci0, ci1, ci2)
            semc = (sc0, sc1, sc2)
        else:
            (srcb, dstb, etb, r0, r1, r2, g0, g1, g2,
             dc0, dc1, dc2, agg_sh, si0, si1, si2,
             sg0, sg1, sg2, sa0, sa1, sa2) = rest
            cidc = semc = None
        rows = (r0, r1, r2)
        gidx = (g0, g1, g2)
        dstc = (dc0, dc1, dc2)
        semi = (si0, si1, si2)
        semg = (sg0, sg1, sg2)
        sema = (sa0, sa1, sa2)
        cid = lax.axis_index("c")
        sid = lax.axis_index("s")
        wid = cid * NS + sid
        base = wid * EPT

        _zero_vmem_rows(r0, CH)

        @pl.loop(sid, NZC, step=NS)
        def _(m):
            pltpu.sync_copy(r0, agg_sh.at[pl.ds(m * CZ, CZ)])
        if with_counts:
            zero16 = jnp.zeros((16,), f32)
            one16 = jnp.ones((16,), f32)

            @pl.loop(0, CC // 16)
            def _(j):
                z1d[pl.ds(j * 16, 16)] = zero16

            @pl.loop(0, CH // 16)
            def _(j):
                ones[pl.ds(j * 16, 16)] = one16

            @pl.loop(sid, NCC, step=NS)
            def _(m):
                pltpu.sync_copy(z1d, cnt_sh.at[pl.ds(m * CC, CC)])
        plsc.subcore_barrier()

        def fire_idx(j, b):
            sl = pl.ds(base + j * CH, CH)
            pltpu.async_copy(src_hbm.at[sl], srcb.at[b], semi[b])
            pltpu.async_copy(dst_hbm.at[sl], dstb.at[b], semi[b])
            pltpu.async_copy(et_hbm.at[sl], etb.at[b], semi[b])

        def wait_idx(b):
            hsl = pl.ds(0, CH)
            pltpu.make_async_copy(src_hbm.at[hsl], srcb.at[b],
                                  semi[b]).wait()
            pltpu.make_async_copy(src_hbm.at[hsl], dstb.at[b],
                                  semi[b]).wait()
            pltpu.make_async_copy(src_hbm.at[hsl], etb.at[b],
                                  semi[b]).wait()

        def build_idx(b):
            @pl.loop(0, CH // 16)
            def _(k):
                sl = pl.ds(k * 16, 16)
                e16 = etb[b, sl]
                gidx[b][sl] = e16 * N + srcb[b, sl]
                dstc[b][sl] = dstb[b, sl]
                if with_counts:
                    cidc[b][sl] = dstb[b, sl] * R + e16

        def fire_gather(j, b):
            pltpu.async_copy(t_hbm.at[gidx[b]], rows[b], semg[b])

        def wait_gather(b):
            pltpu.make_async_copy(t_hbm.at[pl.ds(0, CH)], rows[b],
                                  semg[b]).wait()

        def wait_scatter(b):
            pltpu.make_async_copy(rows[b], agg_sh.at[pl.ds(0, CH)],
                                  sema[b]).wait()
            if with_counts:
                pltpu.make_async_copy(ones, cnt_sh.at[pl.ds(0, CH)],
                                      semc[b]).wait()

        def half(jj, b):
            @pl.when(jj < NCH)
            def _():
                nb = (b + 1) % 3

                @pl.when(jj + 2 < NCH)
                def _():
                    fire_idx(jj + 2, (b + 2) % 3)

                @pl.when(jj + 1 < NCH)
                def _():
                    @pl.when(jj >= 2)
                    def _():
                        wait_scatter(nb)
                    wait_idx(nb)
                    build_idx(nb)
                    fire_gather(jj + 1, nb)
                wait_gather(b)
                pltpu.async_copy(rows[b], agg_sh.at[dstc[b]], sema[b],
                                 add=True)
                if with_counts:
                    pltpu.async_copy(ones, cnt_sh.at[cidc[b]], semc[b],
                                     add=True)

        fire_idx(0, 0)
        fire_idx(1, 1)
        wait_idx(0)
        build_idx(0)
        fire_gather(0, 0)

        @pl.loop(0, NCH + 2, step=3)
        def _(j):
            half(j, 0)
            half(j + 1, 1)
            half(j + 2, 2)

        wait_scatter((NCH - 3) % 3)
        wait_scatter((NCH - 2) % 3)
        wait_scatter((NCH - 1) % 3)
        plsc.subcore_barrier()

        @pl.loop(sid, NZC, step=NS)
        def _(m):
            pltpu.sync_copy(agg_sh.at[pl.ds(m * CZ, CZ)],
                            agg_hbm.at[cid, pl.ds(m * CZ, CZ)])
        if with_counts:
            @pl.loop(sid, NCC, step=NS)
            def _(m):
                pltpu.sync_copy(cnt_sh.at[pl.ds(m * CC, CC)],
                                cnt_hbm.at[cid, pl.ds(m * CC, CC)])

    return agg


_agg_counts = _make_agg(True)
_agg_plain = _make_agg(False)


@functools.partial(
    pl.kernel,
    out_type=(jax.ShapeDtypeStruct((E,), f32), jax.ShapeDtypeStruct((E,), f32)),
    mesh=_MESH,
    scratch_types=(
        pltpu.VMEM((EPT,), i32),
        pltpu.VMEM((EPT,), i32),
        pltpu.VMEM((EPT,), i32),
        pltpu.VMEM((EPT,), i32),
        pltpu.VMEM((CH, H), f32),
        pltpu.VMEM((CH, H), f32),
        pltpu.VMEM((CH, H), f32),
        pltpu.VMEM((CH, H), f32),
        pltpu.VMEM((CH, H), f32),
        pltpu.VMEM((CH, H), f32),
        pltpu.VMEM((EPT,), f32),
        pltpu.VMEM((EPT,), f32),
        pltpu.VMEM((16, 16), f32),
        pltpu.VMEM((16, 16), f32),
        pltpu.SemaphoreType.DMA,
        pltpu.SemaphoreType.DMA,
    ),
    compiler_params=_SC_PARAMS,
)
def _score(tab_hbm, src_hbm, dst_hbm, et_hbm, nds_hbm, pos_hbm, neg_hbm,
           uidxf, didxf, etf, nidxf, urows0, urows1, drows0, drows1,
           nrows0, nrows1, posv, negv, tpbuf, tnbuf, sem0, sem1):
    cid = lax.axis_index("c")
    sid = lax.axis_index("s")
    wid = cid * NS + sid
    base = wid * EPT

    pltpu.sync_copy(src_hbm.at[pl.ds(base, EPT)], uidxf)
    pltpu.sync_copy(dst_hbm.at[pl.ds(base, EPT)], didxf)
    pltpu.sync_copy(et_hbm.at[pl.ds(base, EPT)], etf)
    pltpu.sync_copy(nds_hbm.at[pl.ds(base, EPT)], nidxf)

    @pl.loop(0, EPT // 16)
    def _(k):
        sl = pl.ds(k * 16, 16)
        uidxf[sl] = etf[sl] * N + uidxf[sl]
        didxf[sl] = didxf[sl] + R * N
        nidxf[sl] = nidxf[sl] + R * N

    def fire(j, ur, dr, nr, sem):
        sl = pl.ds(j * CH, CH)
        pltpu.async_copy(tab_hbm.at[uidxf.at[sl]], ur, sem)
        pltpu.async_copy(tab_hbm.at[didxf.at[sl]], dr, sem)
        pltpu.async_copy(tab_hbm.at[nidxf.at[sl]], nr, sem)

    def wait3(ur, dr, nr, sem):
        hsl = pl.ds(0, CH)
        pltpu.make_async_copy(tab_hbm.at[hsl], ur, sem).wait()
        pltpu.make_async_copy(tab_hbm.at[hsl], dr, sem).wait()
        pltpu.make_async_copy(tab_hbm.at[hsl], nr, sem).wait()

    def compute(j, ur, dr, nr):
        rows16 = lax.iota(i32, 16)

        @pl.loop(0, CH // 16)
        def _(g):
            for e in range(16):
                row = g * 16 + e
                p0 = p1 = n0 = n1 = jnp.zeros((16,), f32)
                for cb in range(H // 16):
                    sl = pl.ds(cb * 16, 16)
                    u = ur[row, sl]
                    dd = dr[row, sl]
                    nn = nr[row, sl]
                    if cb % 2 == 0:
                        p0 = p0 + u * dd
                        n0 = n0 + u * nn
                    else:
                        p1 = p1 + u * dd
                        n1 = n1 + u * nn
                tpbuf[e, pl.ds(0, 16)] = p0 + p1
                tnbuf[e, pl.ds(0, 16)] = n0 + n1
            ps0 = ps1 = ns0 = ns1 = jnp.zeros((16,), f32)
            for l in range(16):
                coll = jnp.full((16,), l, i32)
                if l % 2 == 0:
                    ps0 = ps0 + plsc.load_gather(tpbuf, [rows16, coll])
                    ns0 = ns0 + plsc.load_gather(tnbuf, [rows16, coll])
                else:
                    ps1 = ps1 + plsc.load_gather(tpbuf, [rows16, coll])
                    ns1 = ns1 + plsc.load_gather(tnbuf, [rows16, coll])
            posv[pl.ds(j * CH + g * 16, 16)] = ps0 + ps1
            negv[pl.ds(j * CH + g * 16, 16)] = ns0 + ns1

    fire(0, urows0, drows0, nrows0, sem0)

    @pl.loop(0, NCH + 1, step=2)
    def _(j):
        @pl.when(j + 1 < NCH)
        def _():
            fire(j + 1, urows1, drows1, nrows1, sem1)
        wait3(urows0, drows0, nrows0, sem0)
        compute(j, urows0, drows0, nrows0)

        @pl.when(j + 1 < NCH)
        def _():
            @pl.when(j + 2 < NCH)
            def _():
                fire(j + 2, urows0, drows0, nrows0, sem0)
            wait3(urows1, drows1, nrows1, sem1)
            compute(j + 1, urows1, drows1, nrows1)

    pltpu.sync_copy(posv, pos_hbm.at[pl.ds(base, EPT)])
    pltpu.sync_copy(negv, neg_hbm.at[pl.ds(base, EPT)])


def kernel(x, edge_index, edge_type, neg_dst, W_emb, b_emb,
           bases0, w_coe0, self_loop0, gamma0, beta0, mm0, mv0,
           bases1, w_coe1, self_loop1, gamma1, beta1, mm1, mv1,
           w_relation):
    src = edge_index[0]
    dst = edge_index[1]
    row = lambda a: a.reshape(1, H)

    w0 = jnp.concatenate(
        [jnp.einsum('ab,bcd->acd', w_coe0, bases0), self_loop0[None]], 0)
    w1 = jnp.concatenate(
        [jnp.einsum('ab,bcd->acd', w_coe1, bases1), self_loop1[None]], 0)
    w0f = jnp.einsum('cd,rde->rce', W_emb, w0)
    b0f = jnp.einsum('d,rde->re', b_emb, w0)[:, None, :]

    t0 = _proj(x, w0f, b0f)
    aggp0, cnt = _agg_counts(t0.reshape((R + 1) * N, H), src, dst, edge_type)
    t1, norm = _postproj(aggp0, t0[R], cnt.reshape(NC, N, R),
                         row(gamma0), row(beta0), row(mm0), row(mv0), w1)
    (aggp1,) = _agg_plain(t1.reshape((R + 1) * N, H), src, dst, edge_type)
    tab = _post1(aggp1, t1[R], norm,
                 row(gamma1), row(beta1), row(mm1), row(mv1), w_relation)
    pos, neg = _score(tab.reshape((R + 1) * N, H),
                      src, dst, edge_type, neg_dst)
    return pos, neg


# speedup vs baseline: 26.3389x; 1.0016x over previous
import dataclasses
import functools

import jax
import jax.numpy as jnp
from jax import lax
from jax.experimental import pallas as pl
from jax.experimental.pallas import tpu as pltpu
from jax.experimental.pallas import tpu_sc as plsc

N = 10000
E = 320000
H = 128
R = 8
NUM_B = 4
EPS = 1e-3

NC = 2
NS = 16
NW = NC * NS
EPT = E // NW
CH = 80
NCH = EPT // CH
CZ = 80
NZC = N // CZ
CC = 640
NCC = N * R // CC

f32 = jnp.float32
i32 = jnp.int32

_MESH = plsc.VectorSubcoreMesh(core_axis_name="c", subcore_axis_name="s")

_SC_PARAMS = pltpu.CompilerParams()
if "needs_layout_passes" in pltpu.CompilerParams.__dataclass_fields__:
    _SC_PARAMS = dataclasses.replace(_SC_PARAMS, needs_layout_passes=False)


_BLK = 2000


def _proj_body(x_ref, we_ref, be_ref, w_ref, o_ref):
    wf = jnp.dot(we_ref[...], w_ref[0], preferred_element_type=f32)
    bf = jnp.dot(be_ref[...], w_ref[0], preferred_element_type=f32)
    o_ref[0] = jnp.dot(x_ref[...], wf, preferred_element_type=f32) + bf


def _proj(x, wemb, bemb, wstack):
    return pl.pallas_call(
        _proj_body,
        grid=(R + 1, N // _BLK),
        in_specs=[pl.BlockSpec((_BLK, H), lambda r, i: (i, 0)),
                  pl.BlockSpec((H, H), lambda r, i: (0, 0)),
                  pl.BlockSpec((1, H), lambda r, i: (0, 0)),
                  pl.BlockSpec((1, H, H), lambda r, i: (r, 0, 0))],
        out_specs=pl.BlockSpec((1, _BLK, H), lambda r, i: (r, i, 0)),
        out_shape=jax.ShapeDtypeStruct((R + 1, N, H), f32),
    )(x, wemb, bemb, wstack)


def _bn_relu(v, mm, scale, bet):
    return jnp.maximum((v - mm) * scale + bet, 0.0)


def _postproj_body(aggp_ref, hsl_ref, cnt_ref, gam_ref, bet_ref, mm_ref,
                   mv_ref, w1_ref, t_ref, norm_ref):
    ctot = cnt_ref[0] + cnt_ref[1]
    norm = jnp.zeros((_BLK, 1), f32)
    for et in range(R):
        c = ctot[:, et:et + 1]
        norm = jnp.where(c > 0, 1.0 / jnp.maximum(c, 1.0), norm)
    v = (aggp_ref[0] + aggp_ref[1]) * norm + hsl_ref[...]
    scale = gam_ref[...] * lax.rsqrt(mv_ref[...] + EPS)
    h1 = _bn_relu(v, mm_ref[...], scale, bet_ref[...])
    for r in range(R + 1):
        t_ref[r] = jnp.dot(h1, w1_ref[r], preferred_element_type=f32)
    norm_ref[...] = norm


def _postproj(aggp, hsl, cnt, gam, bet, mm, mv, w1):
    row = lambda i: (0, i, 0)
    vec = lambda i: (0, 0)
    return pl.pallas_call(
        _postproj_body,
        grid=(N // _BLK,),
        in_specs=[pl.BlockSpec((NC, _BLK, H), row),
                  pl.BlockSpec((_BLK, H), lambda i: (i, 0)),
                  pl.BlockSpec((NC, _BLK, R), row),
                  pl.BlockSpec((1, H), vec),
                  pl.BlockSpec((1, H), vec),
                  pl.BlockSpec((1, H), vec),
                  pl.BlockSpec((1, H), vec),
                  pl.BlockSpec((R + 1, H, H), lambda i: (0, 0, 0))],
        out_specs=[pl.BlockSpec((R + 1, _BLK, H), lambda i: (0, i, 0)),
                   pl.BlockSpec((_BLK, 1), lambda i: (i, 0))],
        out_shape=[jax.ShapeDtypeStruct((R + 1, N, H), f32),
                   jax.ShapeDtypeStruct((N, 1), f32)],
    )(aggp, hsl, cnt, gam, bet, mm, mv, w1)


def _post1_body(aggp_ref, hsl_ref, norm_ref, gam_ref, bet_ref, mm_ref, mv_ref,
                wrel_ref, tab_ref):
    v = (aggp_ref[0] + aggp_ref[1]) * norm_ref[...] + hsl_ref[...]
    scale = gam_ref[...] * lax.rsqrt(mv_ref[...] + EPS)
    h2 = _bn_relu(v, mm_ref[...], scale, bet_ref[...])
    for r in range(R):
        tab_ref[r] = h2 * wrel_ref[r]
    tab_ref[R] = h2


def _post1(aggp, hsl, norm, gam, bet, mm, mv, wrel):
    vec = lambda i: (0, 0)
    return pl.pallas_call(
        _post1_body,
        grid=(N // _BLK,),
        in_specs=[pl.BlockSpec((NC, _BLK, H), lambda i: (0, i, 0)),
                  pl.BlockSpec((_BLK, H), lambda i: (i, 0)),
                  pl.BlockSpec((_BLK, 1), lambda i: (i, 0)),
                  pl.BlockSpec((1, H), vec),
                  pl.BlockSpec((1, H), vec),
                  pl.BlockSpec((1, H), vec),
                  pl.BlockSpec((1, H), vec),
                  pl.BlockSpec((R, H), vec)],
        out_specs=pl.BlockSpec((R + 1, _BLK, H), lambda i: (0, i, 0)),
        out_shape=jax.ShapeDtypeStruct((R + 1, N, H), f32),
    )(aggp, hsl, norm, gam, bet, mm, mv, wrel)


def _zero_vmem_rows(buf, rows):
    zero16 = jnp.zeros((16,), f32)

    @pl.loop(0, rows)
    def _(r):
        @pl.loop(0, H // 16)
        def _(j):
            buf[r, pl.ds(j * 16, 16)] = zero16


def _make_agg(with_counts):
    out_type = [jax.ShapeDtypeStruct((NC, N, H), f32)]
    scratch = [
        pltpu.VMEM((3, CH), i32),
        pltpu.VMEM((3, CH), i32),
        pltpu.VMEM((3, CH), i32),
        pltpu.VMEM((CH, H), f32),
        pltpu.VMEM((CH, H), f32),
        pltpu.VMEM((CH, H), f32),
        pltpu.VMEM((CH,), i32),
        pltpu.VMEM((CH,), i32),
        pltpu.VMEM((CH,), i32),
        pltpu.VMEM((CH,), i32),
        pltpu.VMEM((CH,), i32),
        pltpu.VMEM((CH,), i32),
        pltpu.VMEM_SHARED((N, H), f32),
        pltpu.SemaphoreType.DMA,
        pltpu.SemaphoreType.DMA,
        pltpu.SemaphoreType.DMA,
        pltpu.SemaphoreType.DMA,
        pltpu.SemaphoreType.DMA,
        pltpu.SemaphoreType.DMA,
        pltpu.SemaphoreType.DMA,
        pltpu.SemaphoreType.DMA,
        pltpu.SemaphoreType.DMA,
    ]
    if with_counts:
        out_type.append(jax.ShapeDtypeStruct((NC, N * R), f32))
        scratch += [
            pltpu.VMEM((CH,), i32),
            pltpu.VMEM((CH,), i32),
            pltpu.VMEM((CH,), i32),
            pltpu.VMEM((CH,), f32),
            pltpu.VMEM((CC,), f32),
            pltpu.VMEM_SHARED((N * R,), f32),
            pltpu.SemaphoreType.DMA,
            pltpu.SemaphoreType.DMA,
            pltpu.SemaphoreType.DMA,
        ]

    @functools.partial(pl.kernel, out_type=tuple(out_type), mesh=_MESH,
                       scratch_types=tuple(scratch))
    def agg(t_hbm, src_hbm, dst_hbm, et_hbm, agg_hbm, *rest):
        if with_counts:
            (cnt_hbm, srcb, dstb, etb, r0, r1, r2, g0, g1, g2,
             dc0, dc1, dc2, agg_sh, si0, si1, si2,
             sg0, sg1, sg2, sa0, sa1, sa2,
             ci0, ci1, ci2, ones, z1d, cnt_sh, sc0, sc1, sc2) = rest
            cidc = (ci0, ci1, ci2)
            semc = (sc0, sc1, sc2)
        else:
            (srcb, dstb, etb, r0, r1, r2, g0, g1, g2,
             dc0, dc1, dc2, agg_sh, si0, si1, si2,
             sg0, sg1, sg2, sa0, sa1, sa2) = rest
            cidc = semc = None
        rows = (r0, r1, r2)
        gidx = (g0, g1, g2)
        dstc = (dc0, dc1, dc2)
        semi = (si0, si1, si2)
        semg = (sg0, sg1, sg2)
        sema = (sa0, sa1, sa2)
        cid = lax.axis_index("c")
        sid = lax.axis_index("s")
        wid = cid * NS + sid
        base = wid * EPT

        _zero_vmem_rows(r0, CH)

        @pl.loop(sid, NZC, step=NS)
        def _(m):
            pltpu.sync_copy(r0, agg_sh.at[pl.ds(m * CZ, CZ)])
        if with_counts:
            zero16 = jnp.zeros((16,), f32)
            one16 = jnp.ones((16,), f32)

            @pl.loop(0, CC // 16)
            def _(j):
                z1d[pl.ds(j * 16, 16)] = zero16

            @pl.loop(0, CH // 16)
            def _(j):
                ones[pl.ds(j * 16, 16)] = one16

            @pl.loop(sid, NCC, step=NS)
            def _(m):
                pltpu.sync_copy(z1d, cnt_sh.at[pl.ds(m * CC, CC)])
        plsc.subcore_barrier()

        def fire_idx(j, b):
            sl = pl.ds(base + j * CH, CH)
            pltpu.async_copy(src_hbm.at[sl], srcb.at[b], semi[b])
            pltpu.async_copy(dst_hbm.at[sl], dstb.at[b], semi[b])
            pltpu.async_copy(et_hbm.at[sl], etb.at[b], semi[b])

        def wait_idx(b):
            hsl = pl.ds(0, CH)
            pltpu.make_async_copy(src_hbm.at[hsl], srcb.at[b],
                                  semi[b]).wait()
            pltpu.make_async_copy(src_hbm.at[hsl], dstb.at[b],
                                  semi[b]).wait()
            pltpu.make_async_copy(src_hbm.at[hsl], etb.at[b],
                                  semi[b]).wait()

        def build_idx(b):
            @pl.loop(0, CH // 16)
            def _(k):
                sl = pl.ds(k * 16, 16)
                e16 = etb[b, sl]
                gidx[b][sl] = e16 * N + srcb[b, sl]
                dstc[b][sl] = dstb[b, sl]
                if with_counts:
                    cidc[b][sl] = dstb[b, sl] * R + e16

        def fire_gather(j, b):
            pltpu.async_copy(t_hbm.at[gidx[b]], rows[b], semg[b])

        def wait_gather(b):
            pltpu.make_async_copy(t_hbm.at[pl.ds(0, CH)], rows[b],
                                  semg[b]).wait()

        def wait_scatter(b):
            pltpu.make_async_copy(rows[b], agg_sh.at[pl.ds(0, CH)],
                                  sema[b]).wait()
            if with_counts:
                pltpu.make_async_copy(ones, cnt_sh.at[pl.ds(0, CH)],
                                      semc[b]).wait()

        def half(jj, b):
            @pl.when(jj < NCH)
            def _():
                nb = (b + 1) % 3

                @pl.when(jj + 2 < NCH)
                def _():
                    fire_idx(jj + 2, (b + 2) % 3)

                @pl.when(jj + 1 < NCH)
                def _():
                    @pl.when(jj >= 2)
                    def _():
                        wait_scatter(nb)
                    wait_idx(nb)
                    build_idx(nb)
                    fire_gather(jj + 1, nb)
                wait_gather(b)
                pltpu.async_copy(rows[b], agg_sh.at[dstc[b]], sema[b],
                                 add=True)
                if with_counts:
                    pltpu.async_copy(ones, cnt_sh.at[cidc[b]], semc[b],
                                     add=True)

        fire_idx(0, 0)
        fire_idx(1, 1)
        wait_idx(0)
        build_idx(0)
        fire_gather(0, 0)

        @pl.loop(0, NCH + 2, step=3)
        def _(j):
            half(j, 0)
            half(j + 1, 1)
            half(j + 2, 2)

        wait_scatter((NCH - 3) % 3)
        wait_scatter((NCH - 2) % 3)
        wait_scatter((NCH - 1) % 3)
        plsc.subcore_barrier()

        @pl.loop(sid, NZC, step=NS)
        def _(m):
            pltpu.sync_copy(agg_sh.at[pl.ds(m * CZ, CZ)],
                            agg_hbm.at[cid, pl.ds(m * CZ, CZ)])
        if with_counts:
            @pl.loop(sid, NCC, step=NS)
            def _(m):
                pltpu.sync_copy(cnt_sh.at[pl.ds(m * CC, CC)],
                                cnt_hbm.at[cid, pl.ds(m * CC, CC)])

    return agg


_agg_counts = _make_agg(True)
_agg_plain = _make_agg(False)


@functools.partial(
    pl.kernel,
    out_type=(jax.ShapeDtypeStruct((E,), f32), jax.ShapeDtypeStruct((E,), f32)),
    mesh=_MESH,
    scratch_types=(
        pltpu.VMEM((EPT,), i32),
        pltpu.VMEM((EPT,), i32),
        pltpu.VMEM((EPT,), i32),
        pltpu.VMEM((EPT,), i32),
        pltpu.VMEM((CH, H), f32),
        pltpu.VMEM((CH, H), f32),
        pltpu.VMEM((CH, H), f32),
        pltpu.VMEM((CH, H), f32),
        pltpu.VMEM((CH, H), f32),
        pltpu.VMEM((CH, H), f32),
        pltpu.VMEM((EPT,), f32),
        pltpu.VMEM((EPT,), f32),
        pltpu.VMEM((16, 16), f32),
        pltpu.VMEM((16, 16), f32),
        pltpu.SemaphoreType.DMA,
        pltpu.SemaphoreType.DMA,
    ),
    compiler_params=_SC_PARAMS,
)
def _score(tab_hbm, src_hbm, dst_hbm, et_hbm, nds_hbm, pos_hbm, neg_hbm,
           uidxf, didxf, etf, nidxf, urows0, urows1, drows0, drows1,
           nrows0, nrows1, posv, negv, tpbuf, tnbuf, sem0, sem1):
    cid = lax.axis_index("c")
    sid = lax.axis_index("s")
    wid = cid * NS + sid
    base = wid * EPT

    pltpu.sync_copy(src_hbm.at[pl.ds(base, EPT)], uidxf)
    pltpu.sync_copy(dst_hbm.at[pl.ds(base, EPT)], didxf)
    pltpu.sync_copy(et_hbm.at[pl.ds(base, EPT)], etf)
    pltpu.sync_copy(nds_hbm.at[pl.ds(base, EPT)], nidxf)

    @pl.loop(0, EPT // 16)
    def _(k):
        sl = pl.ds(k * 16, 16)
        uidxf[sl] = etf[sl] * N + uidxf[sl]
        didxf[sl] = didxf[sl] + R * N
        nidxf[sl] = nidxf[sl] + R * N

    def fire(j, ur, dr, nr, sem):
        sl = pl.ds(j * CH, CH)
        pltpu.async_copy(tab_hbm.at[uidxf.at[sl]], ur, sem)
        pltpu.async_copy(tab_hbm.at[didxf.at[sl]], dr, sem)
        pltpu.async_copy(tab_hbm.at[nidxf.at[sl]], nr, sem)

    def wait3(ur, dr, nr, sem):
        hsl = pl.ds(0, CH)
        pltpu.make_async_copy(tab_hbm.at[hsl], ur, sem).wait()
        pltpu.make_async_copy(tab_hbm.at[hsl], dr, sem).wait()
        pltpu.make_async_copy(tab_hbm.at[hsl], nr, sem).wait()

    def compute(j, ur, dr, nr):
        rows16 = lax.iota(i32, 16)

        @pl.loop(0, CH // 16)
        def _(g):
            for e in range(16):
                row = g * 16 + e
                p0 = p1 = n0 = n1 = jnp.zeros((16,), f32)
                for cb in range(H // 16):
                    sl = pl.ds(cb * 16, 16)
                    u = ur[row, sl]
                    dd = dr[row, sl]
                    nn = nr[row, sl]
                    if cb % 2 == 0:
                        p0 = p0 + u * dd
                        n0 = n0 + u * nn
                    else:
                        p1 = p1 + u * dd
                        n1 = n1 + u * nn
                tpbuf[e, pl.ds(0, 16)] = p0 + p1
                tnbuf[e, pl.ds(0, 16)] = n0 + n1
            ps0 = ps1 = ns0 = ns1 = jnp.zeros((16,), f32)
            for l in range(16):
                coll = jnp.full((16,), l, i32)
                if l % 2 == 0:
                    ps0 = ps0 + plsc.load_gather(tpbuf, [rows16, coll])
                    ns0 = ns0 + plsc.load_gather(tnbuf, [rows16, coll])
                else:
                    ps1 = ps1 + plsc.load_gather(tpbuf, [rows16, coll])
                    ns1 = ns1 + plsc.load_gather(tnbuf, [rows16, coll])
            posv[pl.ds(j * CH + g * 16, 16)] = ps0 + ps1
            negv[pl.ds(j * CH + g * 16, 16)] = ns0 + ns1

    fire(0, urows0, drows0, nrows0, sem0)

    @pl.loop(0, NCH + 1, step=2)
    def _(j):
        @pl.when(j + 1 < NCH)
        def _():
            fire(j + 1, urows1, drows1, nrows1, sem1)
        wait3(urows0, drows0, nrows0, sem0)
        compute(j, urows0, drows0, nrows0)

        @pl.when(j + 1 < NCH)
        def _():
            @pl.when(j + 2 < NCH)
            def _():
                fire(j + 2, urows0, drows0, nrows0, sem0)
            wait3(urows1, drows1, nrows1, sem1)
            compute(j + 1, urows1, drows1, nrows1)

    pltpu.sync_copy(posv, pos_hbm.at[pl.ds(base, EPT)])
    pltpu.sync_copy(negv, neg_hbm.at[pl.ds(base, EPT)])


def kernel(x, edge_index, edge_type, neg_dst, W_emb, b_emb,
           bases0, w_coe0, self_loop0, gamma0, beta0, mm0, mv0,
           bases1, w_coe1, self_loop1, gamma1, beta1, mm1, mv1,
           w_relation):
    src = edge_index[0]
    dst = edge_index[1]
    row = lambda a: a.reshape(1, H)

    w0 = jnp.concatenate(
        [(w_coe0 @ bases0.reshape(NUM_B, H * H)).reshape(R, H, H),
         self_loop0[None]], 0)
    w1 = jnp.concatenate(
        [(w_coe1 @ bases1.reshape(NUM_B, H * H)).reshape(R, H, H),
         self_loop1[None]], 0)

    t0 = _proj(x, W_emb, row(b_emb), w0)
    aggp0, cnt = _agg_counts(t0.reshape((R + 1) * N, H), src, dst, edge_type)
    t1, norm = _postproj(aggp0, t0[R], cnt.reshape(NC, N, R),
                         row(gamma0), row(beta0), row(mm0), row(mv0), w1)
    (aggp1,) = _agg_plain(t1.reshape((R + 1) * N, H), src, dst, edge_type)
    tab = _post1(aggp1, t1[R], norm,
                 row(gamma1), row(beta1), row(mm1), row(mv1), w_relation)
    pos, neg = _score(tab.reshape((R + 1) * N, H),
                      src, dst, edge_type, neg_dst)
    return pos, neg
